# baseline (speedup 1.0000x reference)
import functools

import jax
import jax.numpy as jnp
from jax import lax
from jax.experimental import pallas as pl
from jax.experimental.pallas import tpu as pltpu
from jax.experimental.pallas import tpu_sc as plsc

N = 10000
E = 320000
H = 128
NG = 64
L = 4

NC = 2
NS = 16
NW = NC * NS
EPT = E // NW
CH = 80
NCHUNK = EPT // CH
RA = 624
TAIL = N - NS * RA
ZR = 104

_MESH = dict(core_axis_name="c", subcore_axis_name="s", num_cores=NC,
             num_subcores=NS)


def _silu(v):
    return v * jax.nn.sigmoid(v)


def _deg_body(dst, out, deg_sh, dst_v, ones_v, zbuf, sem):
    del sem
    c = lax.axis_index("c")
    s = lax.axis_index("s")
    ebase = (c * NS + s) * EPT

    def zb(i, carry):
        for k in range(H // 16):
            zbuf[i, pl.ds(16 * k, 16)] = jnp.zeros((16,), jnp.float32)
        return carry
    lax.fori_loop(0, ZR, zb, 0)

    def ob(i, carry):
        for k in range(H // 16):
            ones_v[i, pl.ds(16 * k, 16)] = jnp.ones((16,), jnp.float32)
        return carry
    lax.fori_loop(0, CH, ob, 0)

    rbase = s * RA
    for j in range(RA // ZR):
        pltpu.sync_copy(zbuf, deg_sh.at[pl.ds(rbase + j * ZR, ZR)])

    @pl.when(s == NS - 1)
    def _():
        pltpu.sync_copy(zbuf.at[pl.ds(0, TAIL)],
                        deg_sh.at[pl.ds(NS * RA, TAIL)])
    plsc.subcore_barrier()

    def chunk(i, carry):
        off = ebase + i * CH
        pltpu.sync_copy(dst.at[pl.ds(off, CH)], dst_v)
        pltpu.sync_copy(ones_v, deg_sh.at[dst_v], add=True)
        return carry
    lax.fori_loop(0, NCHUNK, chunk, 0)
    plsc.subcore_barrier()
    pltpu.sync_copy(deg_sh.at[pl.ds(rbase, RA)], out.at[c, pl.ds(rbase, RA)])

    @pl.when(s == NS - 1)
    def _():
        pltpu.sync_copy(deg_sh.at[pl.ds(NS * RA, TAIL)],
                        out.at[c, pl.ds(NS * RA, TAIL)])


def _deg_call(dst):
    f = pl.kernel(
        _deg_body,
        out_type=jax.ShapeDtypeStruct((NC, N, H), jnp.float32),
        mesh=plsc.VectorSubcoreMesh(**_MESH),
        scratch_types=[
            pltpu.VMEM_SHARED((N, H), jnp.float32),
            pltpu.VMEM((CH,), jnp.int32),
            pltpu.VMEM((CH, H), jnp.float32),
            pltpu.VMEM((ZR, H), jnp.float32),
            pltpu.SemaphoreType.DMA,
        ],
    )
    return f(dst)


def _edge_body(P, src, dst, ea, w1, out, S_sh, src_db, dst_db, ea_db, rows_v,
               w1_v, semG0, semG1, semI0, semI1):
    c = lax.axis_index("c")
    s = lax.axis_index("s")
    tile = c * NS + s

    def zb(i, carry):
        for k in range(8):
            rows_v[0, i, pl.ds(16 * k, 16)] = jnp.zeros((16,), jnp.float32)
        return carry
    lax.fori_loop(0, CH, zb, 0)

    rbase = s * RA
    for j in range(7):
        pltpu.sync_copy(rows_v.at[0], S_sh.at[pl.ds(rbase + j * CH, CH)])
    pltpu.sync_copy(rows_v.at[0, pl.ds(0, RA - 7 * CH)],
                    S_sh.at[pl.ds(rbase + 7 * CH, RA - 7 * CH)])

    @pl.when(s == NS - 1)
    def _():
        pltpu.sync_copy(rows_v.at[0, pl.ds(0, TAIL)],
                        S_sh.at[pl.ds(NS * RA, TAIL)])
    plsc.subcore_barrier()

    pltpu.sync_copy(w1, w1_v)
    w1c = [w1_v[pl.ds(16 * k, 16)] for k in range(8)]

    semI = (semI0, semI1)
    semG = (semG0, semG1)

    def idx_start(i, b):
        pltpu.async_copy(src.at[tile, i], src_db.at[b], semI[b])
        pltpu.async_copy(dst.at[tile, i], dst_db.at[b], semI[b])
        pltpu.async_copy(ea.at[tile, i], ea_db.at[b], semI[b])

    def idx_wait(i, b):
        pltpu.make_async_copy(src.at[tile, i], src_db.at[b], semI[b]).wait()
        pltpu.make_async_copy(dst.at[tile, i], dst_db.at[b], semI[b]).wait()
        pltpu.make_async_copy(ea.at[tile, i], ea_db.at[b], semI[b]).wait()

    def g_start(b):
        pltpu.async_copy(P.at[src_db.at[b, 0]], rows_v.at[b], semG[b])

    def g_wait(b):
        pltpu.make_async_copy(P.at[src_db.at[b, 0]], rows_v.at[b],
                              semG[b]).wait()

    def proc(b):
        def grp(g, gcarry):
            eag = ea_db[b, 0, pl.ds(g * 16, 16)]
            for j in range(16):
                eav = jnp.broadcast_to(eag[j], (16,))
                r = g * 16 + j
                for k in range(8):
                    v = rows_v[b, r, pl.ds(16 * k, 16)] + eav * w1c[k]
                    rows_v[b, r, pl.ds(16 * k, 16)] = v / (1.0 + jnp.exp(-v))
            return gcarry
        lax.fori_loop(0, CH // 16, grp, 0)
        pltpu.sync_copy(rows_v.at[b], S_sh.at[dst_db.at[b, 0]], add=True)

    def step(i, b, steady):
        if steady:
            idx_wait(i + 1, 1 - b)
            g_start(1 - b)
        g_wait(b)
        proc(b)
        if steady:
            @pl.when(i + 2 < NCHUNK)
            def _():
                idx_start(i + 2, b)

    idx_start(0, 0)
    idx_wait(0, 0)
    g_start(0)
    idx_start(1, 1)

    def pair(p, carry):
        i0 = 2 * p
        step(i0, 0, True)
        step(i0 + 1, 1, True)
        return carry
    lax.fori_loop(0, NCHUNK // 2, pair, 0)
    step(NCHUNK - 1, 0, False)
    plsc.subcore_barrier()
    pltpu.sync_copy(S_sh.at[pl.ds(rbase, RA)], out.at[c, pl.ds(rbase, RA)])

    @pl.when(s == NS - 1)
    def _():
        pltpu.sync_copy(S_sh.at[pl.ds(NS * RA, TAIL)],
                        out.at[c, pl.ds(NS * RA, TAIL)])


def _edge_call(P, src, dst, ea, w1e):
    f = pl.kernel(
        _edge_body,
        out_type=jax.ShapeDtypeStruct((NC, N, H), jnp.float32),
        mesh=plsc.VectorSubcoreMesh(**_MESH),
        scratch_types=[
            pltpu.VMEM_SHARED((N, H), jnp.float32),
            pltpu.VMEM((2, 1, CH), jnp.int32),
            pltpu.VMEM((2, 1, CH), jnp.int32),
            pltpu.VMEM((2, 1, CH), jnp.float32),
            pltpu.VMEM((2, CH, H), jnp.float32),
            pltpu.VMEM((H,), jnp.float32),
            pltpu.SemaphoreType.DMA,
            pltpu.SemaphoreType.DMA,
            pltpu.SemaphoreType.DMA,
            pltpu.SemaphoreType.DMA,
        ],
    )
    return f(P, src, dst, ea, w1e)


def _tc0_body(x, eW1, eb1, eW2, eb2, W1h, b1, degr, h_o, P_o, invd_o, hed_o):
    xa = x[...]
    hh = _silu(jnp.dot(xa, eW1[...], preferred_element_type=jnp.float32)
               + eb1[...])
    hh = jnp.dot(hh, eW2[...], preferred_element_type=jnp.float32) + eb2[...]
    h_o[...] = hh
    P_o[...] = (jnp.dot(hh, W1h[...], preferred_element_type=jnp.float32)
                + b1[...])
    cnt = degr[0, :, 0:1] + degr[1, :, 0:1]
    invd_o[...] = 1.0 / jnp.maximum(cnt, 1.0)
    hed_o[...] = jnp.minimum(cnt, 1.0)


def _tc0_call(x, eW1, eb1, eW2, eb2, W1h, b1, degr):
    f = pl.pallas_call(
        _tc0_body,
        out_shape=[
            jax.ShapeDtypeStruct((N, H), jnp.float32),
            jax.ShapeDtypeStruct((N, H), jnp.float32),
            jax.ShapeDtypeStruct((N, 1), jnp.float32),
            jax.ShapeDtypeStruct((N, 1), jnp.float32),
        ],
    )
    return f(x, eW1, eb1, eW2, eb2, W1h, b1, degr)


def _layer_update(h, Sp, invd, hed, W2, b2, Ut, Ub, ub, g, bb):
    S = Sp[0] + Sp[1]
    aggr = (jnp.dot(S, W2, preferred_element_type=jnp.float32) * invd
            + b2 * hed)
    u = (jnp.dot(h, Ut, preferred_element_type=jnp.float32)
         + jnp.dot(aggr, Ub, preferred_element_type=jnp.float32) + ub)
    mu = jnp.mean(u, axis=0, keepdims=True)
    uc = u - mu
    var = jnp.mean(uc * uc, axis=0, keepdims=True)
    un = uc / jnp.sqrt(var + 1e-5) * g + bb
    return h + _silu(un)


def _tcu_body(h, Sp, invd, hed, W2, b2, Ut, Ub, ub, g, bb, W1n, b1n,
              h_o, P_o):
    hn = _layer_update(h[...], Sp, invd[...], hed[...], W2[...], b2[...],
                       Ut[...], Ub[...], ub[...], g[...], bb[...])
    h_o[...] = hn
    P_o[...] = (jnp.dot(hn, W1n[...], preferred_element_type=jnp.float32)
                + b1n[...])


def _tcu_call(h, Sp, invd, hed, W2, b2, Ut, Ub, ub, g, bb, W1n, b1n):
    f = pl.pallas_call(
        _tcu_body,
        out_shape=[
            jax.ShapeDtypeStruct((N, H), jnp.float32),
            jax.ShapeDtypeStruct((N, H), jnp.float32),
        ],
    )
    return f(h, Sp, invd, hed, W2, b2, Ut, Ub, ub, g, bb, W1n, b1n)


def _tcf_body(h, Sp, invd, hed, W2, b2, Ut, Ub, ub, g, bb,
              prW1, prb1, prW2, prb2, poW1, pob1, poW2, pob2, batch, out_o):
    hn = _layer_update(h[...], Sp, invd[...], hed[...], W2[...], b2[...],
                       Ut[...], Ub[...], ub[...], g[...], bb[...])
    pre = _silu(jnp.dot(hn, prW1[...], preferred_element_type=jnp.float32)
                + prb1[...])
    pre = (jnp.dot(pre, prW2[...], preferred_element_type=jnp.float32)
           + prb2[...])
    gid = lax.broadcasted_iota(jnp.int32, (NG, N), 0)
    oh = (batch[...] == gid).astype(jnp.float32)
    pooled = jnp.dot(oh, pre, preferred_element_type=jnp.float32)
    o = _silu(jnp.dot(pooled, poW1[...], preferred_element_type=jnp.float32)
              + pob1[...])
    out_o[...] = (jnp.dot(o, poW2[...], preferred_element_type=jnp.float32)
                  + pob2[...])


def _tcf_call(h, Sp, invd, hed, W2, b2, Ut, Ub, ub, g, bb,
              prW1, prb1, prW2, prb2, poW1, pob1, poW2, pob2, batch2d):
    f = pl.pallas_call(
        _tcf_body,
        out_shape=jax.ShapeDtypeStruct((NG, H), jnp.float32),
    )
    return f(h, Sp, invd, hed, W2, b2, Ut, Ub, ub, g, bb,
             prW1, prb1, prW2, prb2, poW1, pob1, poW2, pob2, batch2d)


def kernel(x, edge_index, edge_attr, batch, emb_W1, emb_b1, emb_W2, emb_b2,
           msg_W1, msg_b1, msg_W2, msg_b2, upd_W, upd_b, bn_g, bn_b,
           pre_W1, pre_b1, pre_W2, pre_b2, post_W1, post_b1, post_W2,
           post_b2):
    src = edge_index[0]
    dst = edge_index[1]
    ea = edge_attr.reshape(E)
    src_r = src.reshape(NW, NCHUNK, 1, CH)
    dst_r = dst.reshape(NW, NCHUNK, 1, CH)
    ea_r = ea.reshape(NW, NCHUNK, 1, CH)
    batch2d = batch.reshape(1, N)

    r1 = lambda v: v.reshape(1, -1)

    degr = _deg_call(dst)
    h, P, invd, hed = _tc0_call(
        x, emb_W1, r1(emb_b1), emb_W2, r1(emb_b2),
        msg_W1[0, :H, :], r1(msg_b1[0]), degr)

    for l in range(L):
        Sp = _edge_call(P, src_r, dst_r, ea_r, msg_W1[l, H, :])
        args = (h, Sp, invd, hed, msg_W2[l], r1(msg_b2[l]),
                upd_W[l, :H, :], upd_W[l, H:, :], r1(upd_b[l]),
                r1(bn_g[l]), r1(bn_b[l]))
        if l < L - 1:
            h, P = _tcu_call(*args, msg_W1[l + 1, :H, :], r1(msg_b1[l + 1]))
        else:
            out = _tcf_call(*args, pre_W1, r1(pre_b1), pre_W2, r1(pre_b2),
                            post_W1, r1(post_b1), post_W2, r1(post_b2),
                            batch2d)
    return out


# speedup vs baseline: 2.3572x; 2.3572x over previous
import functools

import jax
import jax.numpy as jnp
from jax import lax
from jax.experimental import pallas as pl
from jax.experimental.pallas import tpu as pltpu
from jax.experimental.pallas import tpu_sc as plsc

N = 10000
E = 320000
H = 128
NG = 64
L = 4

NC = 2
NS = 16
NW = NC * NS
EPT = E // NW
CH = 80
NCHUNK = EPT // CH
RA = 624
TAIL = N - NS * RA
ZR = 104

_MESH = dict(core_axis_name="c", subcore_axis_name="s", num_cores=NC,
             num_subcores=NS)


def _silu(v):
    return v * jax.nn.sigmoid(v)


def _deg_body(dst, out, deg_sh, dst_v, ones_v, zbuf, sem):
    del sem
    c = lax.axis_index("c")
    s = lax.axis_index("s")
    ebase = (c * NS + s) * EPT

    def zb(i, carry):
        for k in range(H // 16):
            zbuf[i, pl.ds(16 * k, 16)] = jnp.zeros((16,), jnp.float32)
        return carry
    lax.fori_loop(0, ZR, zb, 0)

    def ob(i, carry):
        for k in range(H // 16):
            ones_v[i, pl.ds(16 * k, 16)] = jnp.ones((16,), jnp.float32)
        return carry
    lax.fori_loop(0, CH, ob, 0)

    rbase = s * RA
    for j in range(RA // ZR):
        pltpu.sync_copy(zbuf, deg_sh.at[pl.ds(rbase + j * ZR, ZR)])

    @pl.when(s == NS - 1)
    def _():
        pltpu.sync_copy(zbuf.at[pl.ds(0, TAIL)],
                        deg_sh.at[pl.ds(NS * RA, TAIL)])
    plsc.subcore_barrier()

    def chunk(i, carry):
        off = ebase + i * CH
        pltpu.sync_copy(dst.at[pl.ds(off, CH)], dst_v)
        pltpu.sync_copy(ones_v, deg_sh.at[dst_v], add=True)
        return carry
    lax.fori_loop(0, NCHUNK, chunk, 0)
    plsc.subcore_barrier()
    pltpu.sync_copy(deg_sh.at[pl.ds(rbase, RA)], out.at[c, pl.ds(rbase, RA)])

    @pl.when(s == NS - 1)
    def _():
        pltpu.sync_copy(deg_sh.at[pl.ds(NS * RA, TAIL)],
                        out.at[c, pl.ds(NS * RA, TAIL)])


def _deg_call(dst):
    f = pl.kernel(
        _deg_body,
        out_type=jax.ShapeDtypeStruct((NC, N, H), jnp.float32),
        mesh=plsc.VectorSubcoreMesh(**_MESH),
        scratch_types=[
            pltpu.VMEM_SHARED((N, H), jnp.float32),
            pltpu.VMEM((CH,), jnp.int32),
            pltpu.VMEM((CH, H), jnp.float32),
            pltpu.VMEM((ZR, H), jnp.float32),
            pltpu.SemaphoreType.DMA,
        ],
    )
    return f(dst)


def _gather_body(P, src, out, src_db, rows_v, semG0, semG1, semI0, semI1):
    c = lax.axis_index("c")
    s = lax.axis_index("s")
    tile = c * NS + s
    semI = (semI0, semI1)
    semG = (semG0, semG1)

    def idx_start(i, b):
        pltpu.async_copy(src.at[tile, i], src_db.at[b], semI[b])

    def idx_wait(i, b):
        pltpu.make_async_copy(src.at[tile, i], src_db.at[b], semI[b]).wait()

    def g_start(b):
        pltpu.async_copy(P.at[src_db.at[b, 0]], rows_v.at[b], semG[b])

    def g_wait(b):
        pltpu.make_async_copy(P.at[src_db.at[b, 0]], rows_v.at[b],
                              semG[b]).wait()

    def wout(i, b):
        pltpu.sync_copy(rows_v.at[b],
                        out.at[pl.ds(tile * EPT + i * CH, CH)])

    def step(i, b, steady):
        if steady:
            idx_wait(i + 1, 1 - b)
            g_start(1 - b)
        g_wait(b)
        wout(i, b)
        if steady:
            @pl.when(i + 2 < NCHUNK)
            def _():
                idx_start(i + 2, b)

    idx_start(0, 0)
    idx_wait(0, 0)
    g_start(0)
    idx_start(1, 1)

    def pair(p_, carry):
        step(2 * p_, 0, True)
        step(2 * p_ + 1, 1, True)
        return carry
    lax.fori_loop(0, NCHUNK // 2, pair, 0)
    step(NCHUNK - 1, 0, False)


def _gather_call(P, src):
    f = pl.kernel(
        _gather_body,
        out_type=jax.ShapeDtypeStruct((E, H), jnp.float32),
        mesh=plsc.VectorSubcoreMesh(**_MESH),
        scratch_types=[
            pltpu.VMEM((2, 1, CH), jnp.int32),
            pltpu.VMEM((2, CH, H), jnp.float32),
            pltpu.SemaphoreType.DMA,
            pltpu.SemaphoreType.DMA,
            pltpu.SemaphoreType.DMA,
            pltpu.SemaphoreType.DMA,
        ],
    )
    return f(P, src)


def _scatter_body(sE, dst, out, S_sh, dst_db, rows_v, semR0, semR1,
                  semI0, semI1):
    c = lax.axis_index("c")
    s = lax.axis_index("s")
    tile = c * NS + s

    def zb(i, carry):
        for k in range(8):
            rows_v[0, i, pl.ds(16 * k, 16)] = jnp.zeros((16,), jnp.float32)
        return carry
    lax.fori_loop(0, CH, zb, 0)

    rbase = s * RA
    for j in range(7):
        pltpu.sync_copy(rows_v.at[0], S_sh.at[pl.ds(rbase + j * CH, CH)])
    pltpu.sync_copy(rows_v.at[0, pl.ds(0, RA - 7 * CH)],
                    S_sh.at[pl.ds(rbase + 7 * CH, RA - 7 * CH)])

    @pl.when(s == NS - 1)
    def _():
        pltpu.sync_copy(rows_v.at[0, pl.ds(0, TAIL)],
                        S_sh.at[pl.ds(NS * RA, TAIL)])
    plsc.subcore_barrier()

    semI = (semI0, semI1)
    semR = (semR0, semR1)

    def idx_start(i, b):
        pltpu.async_copy(dst.at[tile, i], dst_db.at[b], semI[b])

    def idx_wait(i, b):
        pltpu.make_async_copy(dst.at[tile, i], dst_db.at[b], semI[b]).wait()

    def r_start(i, b):
        pltpu.async_copy(sE.at[pl.ds(tile * EPT + i * CH, CH)],
                         rows_v.at[b], semR[b])

    def r_wait(i, b):
        pltpu.make_async_copy(sE.at[pl.ds(tile * EPT + i * CH, CH)],
                              rows_v.at[b], semR[b]).wait()

    def proc(b):
        pltpu.sync_copy(rows_v.at[b], S_sh.at[dst_db.at[b, 0]], add=True)

    def step(i, b, steady):
        if steady:
            idx_wait(i + 1, 1 - b)
            r_start(i + 1, 1 - b)
        r_wait(i, b)
        proc(b)
        if steady:
            @pl.when(i + 2 < NCHUNK)
            def _():
                idx_start(i + 2, b)

    idx_start(0, 0)
    idx_wait(0, 0)
    r_start(0, 0)
    idx_start(1, 1)

    def pair(p_, carry):
        step(2 * p_, 0, True)
        step(2 * p_ + 1, 1, True)
        return carry
    lax.fori_loop(0, NCHUNK // 2, pair, 0)
    step(NCHUNK - 1, 0, False)
    plsc.subcore_barrier()
    pltpu.sync_copy(S_sh.at[pl.ds(rbase, RA)], out.at[c, pl.ds(rbase, RA)])

    @pl.when(s == NS - 1)
    def _():
        pltpu.sync_copy(S_sh.at[pl.ds(NS * RA, TAIL)],
                        out.at[c, pl.ds(NS * RA, TAIL)])


def _scatter_call(sE, dst):
    f = pl.kernel(
        _scatter_body,
        out_type=jax.ShapeDtypeStruct((NC, N, H), jnp.float32),
        mesh=plsc.VectorSubcoreMesh(**_MESH),
        scratch_types=[
            pltpu.VMEM_SHARED((N, H), jnp.float32),
            pltpu.VMEM((2, 1, CH), jnp.int32),
            pltpu.VMEM((2, CH, H), jnp.float32),
            pltpu.SemaphoreType.DMA,
            pltpu.SemaphoreType.DMA,
            pltpu.SemaphoreType.DMA,
            pltpu.SemaphoreType.DMA,
        ],
    )
    return f(sE, dst)


BLK = 4000


def _silu_body(G, ea, w1, o):
    v = G[...] + ea[...] * w1[...]
    o[...] = v * jax.nn.sigmoid(v)


def _silu_call(G, ea2, w1row):
    f = pl.pallas_call(
        _silu_body,
        grid=(E // BLK,),
        in_specs=[
            pl.BlockSpec((BLK, H), lambda i: (i, 0)),
            pl.BlockSpec((BLK, 1), lambda i: (i, 0)),
            pl.BlockSpec((1, H), lambda i: (0, 0)),
        ],
        out_specs=pl.BlockSpec((BLK, H), lambda i: (i, 0)),
        out_shape=jax.ShapeDtypeStruct((E, H), jnp.float32),
    )
    return f(G, ea2, w1row)


def _tc0_body(x, eW1, eb1, eW2, eb2, W1h, b1, degr, h_o, P_o, invd_o, hed_o):
    xa = x[...]
    hh = _silu(jnp.dot(xa, eW1[...], preferred_element_type=jnp.float32)
               + eb1[...])
    hh = jnp.dot(hh, eW2[...], preferred_element_type=jnp.float32) + eb2[...]
    h_o[...] = hh
    P_o[...] = (jnp.dot(hh, W1h[...], preferred_element_type=jnp.float32)
                + b1[...])
    cnt = degr[0, :, 0:1] + degr[1, :, 0:1]
    invd_o[...] = 1.0 / jnp.maximum(cnt, 1.0)
    hed_o[...] = jnp.minimum(cnt, 1.0)


def _tc0_call(x, eW1, eb1, eW2, eb2, W1h, b1, degr):
    f = pl.pallas_call(
        _tc0_body,
        out_shape=[
            jax.ShapeDtypeStruct((N, H), jnp.float32),
            jax.ShapeDtypeStruct((N, H), jnp.float32),
            jax.ShapeDtypeStruct((N, 1), jnp.float32),
            jax.ShapeDtypeStruct((N, 1), jnp.float32),
        ],
    )
    return f(x, eW1, eb1, eW2, eb2, W1h, b1, degr)


def _layer_update(h, Sp, invd, hed, W2, b2, Ut, Ub, ub, g, bb):
    S = Sp[0] + Sp[1]
    aggr = (jnp.dot(S, W2, preferred_element_type=jnp.float32) * invd
            + b2 * hed)
    u = (jnp.dot(h, Ut, preferred_element_type=jnp.float32)
         + jnp.dot(aggr, Ub, preferred_element_type=jnp.float32) + ub)
    mu = jnp.mean(u, axis=0, keepdims=True)
    uc = u - mu
    var = jnp.mean(uc * uc, axis=0, keepdims=True)
    un = uc / jnp.sqrt(var + 1e-5) * g + bb
    return h + _silu(un)


def _tcu_body(h, Sp, invd, hed, W2, b2, Ut, Ub, ub, g, bb, W1n, b1n,
              h_o, P_o):
    hn = _layer_update(h[...], Sp, invd[...], hed[...], W2[...], b2[...],
                       Ut[...], Ub[...], ub[...], g[...], bb[...])
    h_o[...] = hn
    P_o[...] = (jnp.dot(hn, W1n[...], preferred_element_type=jnp.float32)
                + b1n[...])


def _tcu_call(h, Sp, invd, hed, W2, b2, Ut, Ub, ub, g, bb, W1n, b1n):
    f = pl.pallas_call(
        _tcu_body,
        out_shape=[
            jax.ShapeDtypeStruct((N, H), jnp.float32),
            jax.ShapeDtypeStruct((N, H), jnp.float32),
        ],
    )
    return f(h, Sp, invd, hed, W2, b2, Ut, Ub, ub, g, bb, W1n, b1n)


def _tcf_body(h, Sp, invd, hed, W2, b2, Ut, Ub, ub, g, bb,
              prW1, prb1, prW2, prb2, poW1, pob1, poW2, pob2, batch, out_o):
    hn = _layer_update(h[...], Sp, invd[...], hed[...], W2[...], b2[...],
                       Ut[...], Ub[...], ub[...], g[...], bb[...])
    pre = _silu(jnp.dot(hn, prW1[...], preferred_element_type=jnp.float32)
                + prb1[...])
    pre = (jnp.dot(pre, prW2[...], preferred_element_type=jnp.float32)
           + prb2[...])
    gid = lax.broadcasted_iota(jnp.int32, (NG, N), 0)
    oh = (batch[...] == gid).astype(jnp.float32)
    pooled = jnp.dot(oh, pre, preferred_element_type=jnp.float32)
    o = _silu(jnp.dot(pooled, poW1[...], preferred_element_type=jnp.float32)
              + pob1[...])
    out_o[...] = (jnp.dot(o, poW2[...], preferred_element_type=jnp.float32)
                  + pob2[...])


def _tcf_call(h, Sp, invd, hed, W2, b2, Ut, Ub, ub, g, bb,
              prW1, prb1, prW2, prb2, poW1, pob1, poW2, pob2, batch2d):
    f = pl.pallas_call(
        _tcf_body,
        out_shape=jax.ShapeDtypeStruct((NG, H), jnp.float32),
    )
    return f(h, Sp, invd, hed, W2, b2, Ut, Ub, ub, g, bb,
             prW1, prb1, prW2, prb2, poW1, pob1, poW2, pob2, batch2d)


def kernel(x, edge_index, edge_attr, batch, emb_W1, emb_b1, emb_W2, emb_b2,
           msg_W1, msg_b1, msg_W2, msg_b2, upd_W, upd_b, bn_g, bn_b,
           pre_W1, pre_b1, pre_W2, pre_b2, post_W1, post_b1, post_W2,
           post_b2):
    src = edge_index[0]
    dst = edge_index[1]
    ea = edge_attr.reshape(E)
    src_r = src.reshape(NW, NCHUNK, 1, CH)
    dst_r = dst.reshape(NW, NCHUNK, 1, CH)
    ea2 = ea.reshape(E, 1)
    batch2d = batch.reshape(1, N)

    r1 = lambda v: v.reshape(1, -1)

    degr = _deg_call(dst)
    h, P, invd, hed = _tc0_call(
        x, emb_W1, r1(emb_b1), emb_W2, r1(emb_b2),
        msg_W1[0, :H, :], r1(msg_b1[0]), degr)

    for l in range(L):
        G = _gather_call(P, src_r)
        sE = _silu_call(G, ea2, msg_W1[l, H, :].reshape(1, H))
        Sp = _scatter_call(sE, dst_r)
        args = (h, Sp, invd, hed, msg_W2[l], r1(msg_b2[l]),
                upd_W[l, :H, :], upd_W[l, H:, :], r1(upd_b[l]),
                r1(bn_g[l]), r1(bn_b[l]))
        if l < L - 1:
            h, P = _tcu_call(*args, msg_W1[l + 1, :H, :], r1(msg_b1[l + 1]))
        else:
            out = _tcf_call(*args, pre_W1, r1(pre_b1), pre_W2, r1(pre_b2),
                            post_W1, r1(post_b1), post_W2, r1(post_b2),
                            batch2d)
    return out


# speedup vs baseline: 2.4077x; 1.0214x over previous
import functools

import jax
import jax.numpy as jnp
from jax import lax
from jax.experimental import pallas as pl
from jax.experimental.pallas import tpu as pltpu
from jax.experimental.pallas import tpu_sc as plsc

N = 10000
E = 320000
H = 128
NG = 64
L = 4

NC = 2
NS = 16
NW = NC * NS
EPT = E // NW
CH = 80
NCHUNK = EPT // CH
RA = 624
TAIL = N - NS * RA
ZR = 104

_MESH = dict(core_axis_name="c", subcore_axis_name="s", num_cores=NC,
             num_subcores=NS)


def _silu(v):
    return v * jax.nn.sigmoid(v)


def _deg_body(dst, out, deg_sh, dst_db, ones_v, semI0, semI1):
    c = lax.axis_index("c")
    s = lax.axis_index("s")
    tile = c * NS + s

    def ob(i, carry):
        for k in range(H // 16):
            ones_v[i, pl.ds(16 * k, 16)] = jnp.ones((16,), jnp.float32)
        return carry
    lax.fori_loop(0, CH, ob, 0)

    def zb(i, carry):
        for k in range(H // 16):
            ones_v[CH + i, pl.ds(16 * k, 16)] = jnp.zeros((16,), jnp.float32)
        return carry
    lax.fori_loop(0, CH, zb, 0)

    rbase = s * RA
    for j in range(7):
        pltpu.sync_copy(ones_v.at[pl.ds(CH, CH)],
                        deg_sh.at[pl.ds(rbase + j * CH, CH)])
    pltpu.sync_copy(ones_v.at[pl.ds(CH, RA - 7 * CH)],
                    deg_sh.at[pl.ds(rbase + 7 * CH, RA - 7 * CH)])

    @pl.when(s == NS - 1)
    def _():
        pltpu.sync_copy(ones_v.at[pl.ds(CH, TAIL)],
                        deg_sh.at[pl.ds(NS * RA, TAIL)])
    plsc.subcore_barrier()

    semI = (semI0, semI1)

    def idx_start(i, b):
        pltpu.async_copy(dst.at[tile, i], dst_db.at[b], semI[b])

    def idx_wait(i, b):
        pltpu.make_async_copy(dst.at[tile, i], dst_db.at[b], semI[b]).wait()

    def proc(b):
        pltpu.sync_copy(ones_v.at[pl.ds(0, CH)], deg_sh.at[dst_db.at[b, 0]],
                        add=True)

    def step(i, b, steady):
        idx_wait(i, b)
        if steady:
            @pl.when(i + 1 < NCHUNK)
            def _():
                idx_start(i + 1, 1 - b)
        proc(b)

    idx_start(0, 0)

    def pair(p_, carry):
        step(2 * p_, 0, True)
        step(2 * p_ + 1, 1, True)
        return carry
    lax.fori_loop(0, NCHUNK // 2, pair, 0)
    step(NCHUNK - 1, 0, False)
    plsc.subcore_barrier()
    pltpu.sync_copy(deg_sh.at[pl.ds(rbase, RA)], out.at[c, pl.ds(rbase, RA)])

    @pl.when(s == NS - 1)
    def _():
        pltpu.sync_copy(deg_sh.at[pl.ds(NS * RA, TAIL)],
                        out.at[c, pl.ds(NS * RA, TAIL)])


def _deg_call(dst):
    f = pl.kernel(
        _deg_body,
        out_type=jax.ShapeDtypeStruct((NC, N, H), jnp.float32),
        mesh=plsc.VectorSubcoreMesh(**_MESH),
        scratch_types=[
            pltpu.VMEM_SHARED((N, H), jnp.float32),
            pltpu.VMEM((2, 1, CH), jnp.int32),
            pltpu.VMEM((2 * CH, H), jnp.float32),
            pltpu.SemaphoreType.DMA,
            pltpu.SemaphoreType.DMA,
        ],
    )
    return f(dst)


def _gather_body(P, src, out, src_db, rows_v, semG0, semG1, semI0, semI1):
    c = lax.axis_index("c")
    s = lax.axis_index("s")
    tile = c * NS + s
    semI = (semI0, semI1)
    semG = (semG0, semG1)

    def idx_start(i, b):
        pltpu.async_copy(src.at[tile, i], src_db.at[b], semI[b])

    def idx_wait(i, b):
        pltpu.make_async_copy(src.at[tile, i], src_db.at[b], semI[b]).wait()

    def g_start(b):
        pltpu.async_copy(P.at[src_db.at[b, 0]], rows_v.at[b], semG[b])

    def g_wait(b):
        pltpu.make_async_copy(P.at[src_db.at[b, 0]], rows_v.at[b],
                              semG[b]).wait()

    def wout(i, b):
        pltpu.sync_copy(rows_v.at[b],
                        out.at[pl.ds(tile * EPT + i * CH, CH)])

    def step(i, b, steady):
        if steady:
            idx_wait(i + 1, 1 - b)
            g_start(1 - b)
        g_wait(b)
        wout(i, b)
        if steady:
            @pl.when(i + 2 < NCHUNK)
            def _():
                idx_start(i + 2, b)

    idx_start(0, 0)
    idx_wait(0, 0)
    g_start(0)
    idx_start(1, 1)

    def pair(p_, carry):
        step(2 * p_, 0, True)
        step(2 * p_ + 1, 1, True)
        return carry
    lax.fori_loop(0, NCHUNK // 2, pair, 0)
    step(NCHUNK - 1, 0, False)


def _gather_call(P, src):
    f = pl.kernel(
        _gather_body,
        out_type=jax.ShapeDtypeStruct((E, H), jnp.float32),
        mesh=plsc.VectorSubcoreMesh(**_MESH),
        scratch_types=[
            pltpu.VMEM((2, 1, CH), jnp.int32),
            pltpu.VMEM((2, CH, H), jnp.float32),
            pltpu.SemaphoreType.DMA,
            pltpu.SemaphoreType.DMA,
            pltpu.SemaphoreType.DMA,
            pltpu.SemaphoreType.DMA,
        ],
    )
    return f(P, src)


def _scatter_body(sE, dst, out, S_sh, dst_db, rows_v, semR0, semR1,
                  semI0, semI1):
    c = lax.axis_index("c")
    s = lax.axis_index("s")
    tile = c * NS + s

    def zb(i, carry):
        for k in range(8):
            rows_v[0, i, pl.ds(16 * k, 16)] = jnp.zeros((16,), jnp.float32)
        return carry
    lax.fori_loop(0, CH, zb, 0)

    rbase = s * RA
    for j in range(7):
        pltpu.sync_copy(rows_v.at[0], S_sh.at[pl.ds(rbase + j * CH, CH)])
    pltpu.sync_copy(rows_v.at[0, pl.ds(0, RA - 7 * CH)],
                    S_sh.at[pl.ds(rbase + 7 * CH, RA - 7 * CH)])

    @pl.when(s == NS - 1)
    def _():
        pltpu.sync_copy(rows_v.at[0, pl.ds(0, TAIL)],
                        S_sh.at[pl.ds(NS * RA, TAIL)])
    plsc.subcore_barrier()

    semI = (semI0, semI1)
    semR = (semR0, semR1)

    def idx_start(i, b):
        pltpu.async_copy(dst.at[tile, i], dst_db.at[b], semI[b])

    def idx_wait(i, b):
        pltpu.make_async_copy(dst.at[tile, i], dst_db.at[b], semI[b]).wait()

    def r_start(i, b):
        pltpu.async_copy(sE.at[pl.ds(tile * EPT + i * CH, CH)],
                         rows_v.at[b], semR[b])

    def r_wait(i, b):
        pltpu.make_async_copy(sE.at[pl.ds(tile * EPT + i * CH, CH)],
                              rows_v.at[b], semR[b]).wait()

    def proc(b):
        pltpu.sync_copy(rows_v.at[b], S_sh.at[dst_db.at[b, 0]], add=True)

    def step(i, b, steady):
        if steady:
            idx_wait(i + 1, 1 - b)
            r_start(i + 1, 1 - b)
        r_wait(i, b)
        proc(b)
        if steady:
            @pl.when(i + 2 < NCHUNK)
            def _():
                idx_start(i + 2, b)

    idx_start(0, 0)
    idx_wait(0, 0)
    r_start(0, 0)
    idx_start(1, 1)

    def pair(p_, carry):
        step(2 * p_, 0, True)
        step(2 * p_ + 1, 1, True)
        return carry
    lax.fori_loop(0, NCHUNK // 2, pair, 0)
    step(NCHUNK - 1, 0, False)
    plsc.subcore_barrier()
    pltpu.sync_copy(S_sh.at[pl.ds(rbase, RA)], out.at[c, pl.ds(rbase, RA)])

    @pl.when(s == NS - 1)
    def _():
        pltpu.sync_copy(S_sh.at[pl.ds(NS * RA, TAIL)],
                        out.at[c, pl.ds(NS * RA, TAIL)])


def _scatter_call(sE, dst):
    f = pl.kernel(
        _scatter_body,
        out_type=jax.ShapeDtypeStruct((NC, N, H), jnp.float32),
        mesh=plsc.VectorSubcoreMesh(**_MESH),
        scratch_types=[
            pltpu.VMEM_SHARED((N, H), jnp.float32),
            pltpu.VMEM((2, 1, CH), jnp.int32),
            pltpu.VMEM((2, CH, H), jnp.float32),
            pltpu.SemaphoreType.DMA,
            pltpu.SemaphoreType.DMA,
            pltpu.SemaphoreType.DMA,
            pltpu.SemaphoreType.DMA,
        ],
    )
    return f(sE, dst)


BLK = 4000


def _silu_body(G, ea, w1, o):
    v = G[...] + ea[...] * w1[...]
    o[...] = v * jax.nn.sigmoid(v)


def _silu_call(G, ea2, w1row):
    f = pl.pallas_call(
        _silu_body,
        grid=(E // BLK,),
        in_specs=[
            pl.BlockSpec((BLK, H), lambda i: (i, 0)),
            pl.BlockSpec((BLK, 1), lambda i: (i, 0)),
            pl.BlockSpec((1, H), lambda i: (0, 0)),
        ],
        out_specs=pl.BlockSpec((BLK, H), lambda i: (i, 0)),
        out_shape=jax.ShapeDtypeStruct((E, H), jnp.float32),
    )
    return f(G, ea2, w1row)


def _tc0_body(x, eW1, eb1, eW2, eb2, W1h, b1, degr, h_o, P_o, invd_o, hed_o):
    xa = x[...]
    hh = _silu(jnp.dot(xa, eW1[...], preferred_element_type=jnp.float32)
               + eb1[...])
    hh = jnp.dot(hh, eW2[...], preferred_element_type=jnp.float32) + eb2[...]
    h_o[...] = hh
    P_o[...] = (jnp.dot(hh, W1h[...], preferred_element_type=jnp.float32)
                + b1[...])
    cnt = degr[0, :, 0:1] + degr[1, :, 0:1]
    invd_o[...] = 1.0 / jnp.maximum(cnt, 1.0)
    hed_o[...] = jnp.minimum(cnt, 1.0)


def _tc0_call(x, eW1, eb1, eW2, eb2, W1h, b1, degr):
    f = pl.pallas_call(
        _tc0_body,
        out_shape=[
            jax.ShapeDtypeStruct((N, H), jnp.float32),
            jax.ShapeDtypeStruct((N, H), jnp.float32),
            jax.ShapeDtypeStruct((N, 1), jnp.float32),
            jax.ShapeDtypeStruct((N, 1), jnp.float32),
        ],
    )
    return f(x, eW1, eb1, eW2, eb2, W1h, b1, degr)


def _layer_update(h, Sp, invd, hed, W2, b2, Ut, Ub, ub, g, bb):
    S = Sp[0] + Sp[1]
    aggr = (jnp.dot(S, W2, preferred_element_type=jnp.float32) * invd
            + b2 * hed)
    u = (jnp.dot(h, Ut, preferred_element_type=jnp.float32)
         + jnp.dot(aggr, Ub, preferred_element_type=jnp.float32) + ub)
    mu = jnp.mean(u, axis=0, keepdims=True)
    uc = u - mu
    var = jnp.mean(uc * uc, axis=0, keepdims=True)
    un = uc / jnp.sqrt(var + 1e-5) * g + bb
    return h + _silu(un)


def _tcu_body(h, Sp, invd, hed, W2, b2, Ut, Ub, ub, g, bb, W1n, b1n,
              h_o, P_o):
    hn = _layer_update(h[...], Sp, invd[...], hed[...], W2[...], b2[...],
                       Ut[...], Ub[...], ub[...], g[...], bb[...])
    h_o[...] = hn
    P_o[...] = (jnp.dot(hn, W1n[...], preferred_element_type=jnp.float32)
                + b1n[...])


def _tcu_call(h, Sp, invd, hed, W2, b2, Ut, Ub, ub, g, bb, W1n, b1n):
    f = pl.pallas_call(
        _tcu_body,
        out_shape=[
            jax.ShapeDtypeStruct((N, H), jnp.float32),
            jax.ShapeDtypeStruct((N, H), jnp.float32),
        ],
    )
    return f(h, Sp, invd, hed, W2, b2, Ut, Ub, ub, g, bb, W1n, b1n)


def _tcf_body(h, Sp, invd, hed, W2, b2, Ut, Ub, ub, g, bb,
              prW1, prb1, prW2, prb2, poW1, pob1, poW2, pob2, batch, out_o):
    hn = _layer_update(h[...], Sp, invd[...], hed[...], W2[...], b2[...],
                       Ut[...], Ub[...], ub[...], g[...], bb[...])
    pre = _silu(jnp.dot(hn, prW1[...], preferred_element_type=jnp.float32)
                + prb1[...])
    pre = (jnp.dot(pre, prW2[...], preferred_element_type=jnp.float32)
           + prb2[...])
    gid = lax.broadcasted_iota(jnp.int32, (NG, N), 0)
    oh = (batch[...] == gid).astype(jnp.float32)
    pooled = jnp.dot(oh, pre, preferred_element_type=jnp.float32)
    o = _silu(jnp.dot(pooled, poW1[...], preferred_element_type=jnp.float32)
              + pob1[...])
    out_o[...] = (jnp.dot(o, poW2[...], preferred_element_type=jnp.float32)
                  + pob2[...])


def _tcf_call(h, Sp, invd, hed, W2, b2, Ut, Ub, ub, g, bb,
              prW1, prb1, prW2, prb2, poW1, pob1, poW2, pob2, batch2d):
    f = pl.pallas_call(
        _tcf_body,
        out_shape=jax.ShapeDtypeStruct((NG, H), jnp.float32),
    )
    return f(h, Sp, invd, hed, W2, b2, Ut, Ub, ub, g, bb,
             prW1, prb1, prW2, prb2, poW1, pob1, poW2, pob2, batch2d)


def kernel(x, edge_index, edge_attr, batch, emb_W1, emb_b1, emb_W2, emb_b2,
           msg_W1, msg_b1, msg_W2, msg_b2, upd_W, upd_b, bn_g, bn_b,
           pre_W1, pre_b1, pre_W2, pre_b2, post_W1, post_b1, post_W2,
           post_b2):
    src = edge_index[0]
    dst = edge_index[1]
    ea = edge_attr.reshape(E)
    src_r = src.reshape(NW, NCHUNK, 1, CH)
    dst_r = dst.reshape(NW, NCHUNK, 1, CH)
    ea2 = ea.reshape(E, 1)
    batch2d = batch.reshape(1, N)

    r1 = lambda v: v.reshape(1, -1)

    degr = _deg_call(dst_r)
    h, P, invd, hed = _tc0_call(
        x, emb_W1, r1(emb_b1), emb_W2, r1(emb_b2),
        msg_W1[0, :H, :], r1(msg_b1[0]), degr)

    for l in range(L):
        G = _gather_call(P, src_r)
        sE = _silu_call(G, ea2, msg_W1[l, H, :].reshape(1, H))
        Sp = _scatter_call(sE, dst_r)
        args = (h, Sp, invd, hed, msg_W2[l], r1(msg_b2[l]),
                upd_W[l, :H, :], upd_W[l, H:, :], r1(upd_b[l]),
                r1(bn_g[l]), r1(bn_b[l]))
        if l < L - 1:
            h, P = _tcu_call(*args, msg_W1[l + 1, :H, :], r1(msg_b1[l + 1]))
        else:
            out = _tcf_call(*args, pre_W1, r1(pre_b1), pre_W2, r1(pre_b2),
                            post_W1, r1(post_b1), post_W2, r1(post_b2),
                            batch2d)
    return out


# speedup vs baseline: 2.5467x; 1.0577x over previous
import functools

import jax
import jax.numpy as jnp
from jax import lax
from jax.experimental import pallas as pl
from jax.experimental.pallas import tpu as pltpu
from jax.experimental.pallas import tpu_sc as plsc

N = 10000
E = 320000
H = 128
NG = 64
L = 4

NC = 2
NS = 16
NW = NC * NS
EPT = E // NW
CH = 80
NCHUNK = EPT // CH
RA = 624
TAIL = N - NS * RA
ZR = 104

_MESH = dict(core_axis_name="c", subcore_axis_name="s", num_cores=NC,
             num_subcores=NS)


def _silu(v):
    return v * jax.nn.sigmoid(v)


def _deg_body(dst, out, deg_sh, dst_db, ones_v, semI0, semI1):
    c = lax.axis_index("c")
    s = lax.axis_index("s")
    tile = c * NS + s

    def ob(i, carry):
        for k in range(H // 16):
            ones_v[i, pl.ds(16 * k, 16)] = jnp.ones((16,), jnp.float32)
        return carry
    lax.fori_loop(0, CH, ob, 0)

    def zb(i, carry):
        for k in range(H // 16):
            ones_v[CH + i, pl.ds(16 * k, 16)] = jnp.zeros((16,), jnp.float32)
        return carry
    lax.fori_loop(0, CH, zb, 0)

    rbase = s * RA
    for j in range(7):
        pltpu.sync_copy(ones_v.at[pl.ds(CH, CH)],
                        deg_sh.at[pl.ds(rbase + j * CH, CH)])
    pltpu.sync_copy(ones_v.at[pl.ds(CH, RA - 7 * CH)],
                    deg_sh.at[pl.ds(rbase + 7 * CH, RA - 7 * CH)])

    @pl.when(s == NS - 1)
    def _():
        pltpu.sync_copy(ones_v.at[pl.ds(CH, TAIL)],
                        deg_sh.at[pl.ds(NS * RA, TAIL)])
    plsc.subcore_barrier()

    semI = (semI0, semI1)

    def idx_start(i, b):
        pltpu.async_copy(dst.at[tile, i], dst_db.at[b], semI[b])

    def idx_wait(i, b):
        pltpu.make_async_copy(dst.at[tile, i], dst_db.at[b], semI[b]).wait()

    def proc(b):
        pltpu.sync_copy(ones_v.at[pl.ds(0, CH)], deg_sh.at[dst_db.at[b, 0]],
                        add=True)

    def step(i, b, steady):
        idx_wait(i, b)
        if steady:
            @pl.when(i + 1 < NCHUNK)
            def _():
                idx_start(i + 1, 1 - b)
        proc(b)

    idx_start(0, 0)

    def pair(p_, carry):
        step(2 * p_, 0, True)
        step(2 * p_ + 1, 1, True)
        return carry
    lax.fori_loop(0, NCHUNK // 2, pair, 0)
    step(NCHUNK - 1, 0, False)
    plsc.subcore_barrier()
    pltpu.sync_copy(deg_sh.at[pl.ds(rbase, RA)], out.at[c, pl.ds(rbase, RA)])

    @pl.when(s == NS - 1)
    def _():
        pltpu.sync_copy(deg_sh.at[pl.ds(NS * RA, TAIL)],
                        out.at[c, pl.ds(NS * RA, TAIL)])


def _deg_call(dst):
    f = pl.kernel(
        _deg_body,
        out_type=jax.ShapeDtypeStruct((NC, N, H), jnp.float32),
        mesh=plsc.VectorSubcoreMesh(**_MESH),
        scratch_types=[
            pltpu.VMEM_SHARED((N, H), jnp.float32),
            pltpu.VMEM((2, 1, CH), jnp.int32),
            pltpu.VMEM((2 * CH, H), jnp.float32),
            pltpu.SemaphoreType.DMA,
            pltpu.SemaphoreType.DMA,
        ],
    )
    return f(dst)


def _gather_body(P, src, out, P_sh, src_db, rows_v, semG0, semG1,
                 semI0, semI1):
    c = lax.axis_index("c")
    s = lax.axis_index("s")
    tile = c * NS + s
    semI = (semI0, semI1)
    semG = (semG0, semG1)

    rbase = s * RA
    pltpu.sync_copy(P.at[pl.ds(rbase, RA)], P_sh.at[pl.ds(rbase, RA)])

    @pl.when(s == NS - 1)
    def _():
        pltpu.sync_copy(P.at[pl.ds(NS * RA, TAIL)],
                        P_sh.at[pl.ds(NS * RA, TAIL)])
    plsc.subcore_barrier()

    def idx_start(i, b):
        pltpu.async_copy(src.at[tile, i], src_db.at[b], semI[b])

    def idx_wait(i, b):
        pltpu.make_async_copy(src.at[tile, i], src_db.at[b], semI[b]).wait()

    def g_start(b):
        pltpu.async_copy(P_sh.at[src_db.at[b, 0]], rows_v.at[b], semG[b])

    def g_wait(b):
        pltpu.make_async_copy(P_sh.at[src_db.at[b, 0]], rows_v.at[b],
                              semG[b]).wait()

    def wout(i, b):
        pltpu.sync_copy(rows_v.at[b],
                        out.at[pl.ds(tile * EPT + i * CH, CH)])

    def step(i, b, steady):
        if steady:
            idx_wait(i + 1, 1 - b)
            g_start(1 - b)
        g_wait(b)
        wout(i, b)
        if steady:
            @pl.when(i + 2 < NCHUNK)
            def _():
                idx_start(i + 2, b)

    idx_start(0, 0)
    idx_wait(0, 0)
    g_start(0)
    idx_start(1, 1)

    def pair(p_, carry):
        step(2 * p_, 0, True)
        step(2 * p_ + 1, 1, True)
        return carry
    lax.fori_loop(0, NCHUNK // 2, pair, 0)
    step(NCHUNK - 1, 0, False)


def _gather_call(P, src):
    f = pl.kernel(
        _gather_body,
        out_type=jax.ShapeDtypeStruct((E, H), jnp.float32),
        mesh=plsc.VectorSubcoreMesh(**_MESH),
        scratch_types=[
            pltpu.VMEM_SHARED((N, H), jnp.float32),
            pltpu.VMEM((2, 1, CH), jnp.int32),
            pltpu.VMEM((2, CH, H), jnp.float32),
            pltpu.SemaphoreType.DMA,
            pltpu.SemaphoreType.DMA,
            pltpu.SemaphoreType.DMA,
            pltpu.SemaphoreType.DMA,
        ],
    )
    return f(P, src)


def _scatter_body(sE, dst, out, S_sh, dst_db, rows_v, semR0, semR1,
                  semI0, semI1):
    c = lax.axis_index("c")
    s = lax.axis_index("s")
    tile = c * NS + s

    def zb(i, carry):
        for k in range(8):
            rows_v[0, i, pl.ds(16 * k, 16)] = jnp.zeros((16,), jnp.float32)
        return carry
    lax.fori_loop(0, CH, zb, 0)

    rbase = s * RA
    for j in range(7):
        pltpu.sync_copy(rows_v.at[0], S_sh.at[pl.ds(rbase + j * CH, CH)])
    pltpu.sync_copy(rows_v.at[0, pl.ds(0, RA - 7 * CH)],
                    S_sh.at[pl.ds(rbase + 7 * CH, RA - 7 * CH)])

    @pl.when(s == NS - 1)
    def _():
        pltpu.sync_copy(rows_v.at[0, pl.ds(0, TAIL)],
                        S_sh.at[pl.ds(NS * RA, TAIL)])
    plsc.subcore_barrier()

    semI = (semI0, semI1)
    semR = (semR0, semR1)

    def idx_start(i, b):
        pltpu.async_copy(dst.at[tile, i], dst_db.at[b], semI[b])

    def idx_wait(i, b):
        pltpu.make_async_copy(dst.at[tile, i], dst_db.at[b], semI[b]).wait()

    def r_start(i, b):
        pltpu.async_copy(sE.at[pl.ds(tile * EPT + i * CH, CH)],
                         rows_v.at[b], semR[b])

    def r_wait(i, b):
        pltpu.make_async_copy(sE.at[pl.ds(tile * EPT + i * CH, CH)],
                              rows_v.at[b], semR[b]).wait()

    def proc(b):
        pltpu.sync_copy(rows_v.at[b], S_sh.at[dst_db.at[b, 0]], add=True)

    def step(i, b, steady):
        if steady:
            idx_wait(i + 1, 1 - b)
            r_start(i + 1, 1 - b)
        r_wait(i, b)
        proc(b)
        if steady:
            @pl.when(i + 2 < NCHUNK)
            def _():
                idx_start(i + 2, b)

    idx_start(0, 0)
    idx_wait(0, 0)
    r_start(0, 0)
    idx_start(1, 1)

    def pair(p_, carry):
        step(2 * p_, 0, True)
        step(2 * p_ + 1, 1, True)
        return carry
    lax.fori_loop(0, NCHUNK // 2, pair, 0)
    step(NCHUNK - 1, 0, False)
    plsc.subcore_barrier()
    pltpu.sync_copy(S_sh.at[pl.ds(rbase, RA)], out.at[c, pl.ds(rbase, RA)])

    @pl.when(s == NS - 1)
    def _():
        pltpu.sync_copy(S_sh.at[pl.ds(NS * RA, TAIL)],
                        out.at[c, pl.ds(NS * RA, TAIL)])


def _scatter_call(sE, dst):
    f = pl.kernel(
        _scatter_body,
        out_type=jax.ShapeDtypeStruct((NC, N, H), jnp.float32),
        mesh=plsc.VectorSubcoreMesh(**_MESH),
        scratch_types=[
            pltpu.VMEM_SHARED((N, H), jnp.float32),
            pltpu.VMEM((2, 1, CH), jnp.int32),
            pltpu.VMEM((2, CH, H), jnp.float32),
            pltpu.SemaphoreType.DMA,
            pltpu.SemaphoreType.DMA,
            pltpu.SemaphoreType.DMA,
            pltpu.SemaphoreType.DMA,
        ],
    )
    return f(sE, dst)


BLK = 4000


def _silu_body(G, ea, w1, o):
    v = G[...] + ea[...] * w1[...]
    o[...] = v * jax.nn.sigmoid(v)


def _silu_call(G, ea2, w1row):
    f = pl.pallas_call(
        _silu_body,
        grid=(E // BLK,),
        in_specs=[
            pl.BlockSpec((BLK, H), lambda i: (i, 0)),
            pl.BlockSpec((BLK, 1), lambda i: (i, 0)),
            pl.BlockSpec((1, H), lambda i: (0, 0)),
        ],
        out_specs=pl.BlockSpec((BLK, H), lambda i: (i, 0)),
        out_shape=jax.ShapeDtypeStruct((E, H), jnp.float32),
    )
    return f(G, ea2, w1row)


def _tc0_body(x, eW1, eb1, eW2, eb2, W1h, b1, degr, h_o, P_o, invd_o, hed_o):
    xa = x[...]
    hh = _silu(jnp.dot(xa, eW1[...], preferred_element_type=jnp.float32)
               + eb1[...])
    hh = jnp.dot(hh, eW2[...], preferred_element_type=jnp.float32) + eb2[...]
    h_o[...] = hh
    P_o[...] = (jnp.dot(hh, W1h[...], preferred_element_type=jnp.float32)
                + b1[...])
    cnt = degr[0, :, 0:1] + degr[1, :, 0:1]
    invd_o[...] = 1.0 / jnp.maximum(cnt, 1.0)
    hed_o[...] = jnp.minimum(cnt, 1.0)


def _tc0_call(x, eW1, eb1, eW2, eb2, W1h, b1, degr):
    f = pl.pallas_call(
        _tc0_body,
        out_shape=[
            jax.ShapeDtypeStruct((N, H), jnp.float32),
            jax.ShapeDtypeStruct((N, H), jnp.float32),
            jax.ShapeDtypeStruct((N, 1), jnp.float32),
            jax.ShapeDtypeStruct((N, 1), jnp.float32),
        ],
    )
    return f(x, eW1, eb1, eW2, eb2, W1h, b1, degr)


def _layer_update(h, Sp, invd, hed, W2, b2, Ut, Ub, ub, g, bb):
    S = Sp[0] + Sp[1]
    aggr = (jnp.dot(S, W2, preferred_element_type=jnp.float32) * invd
            + b2 * hed)
    u = (jnp.dot(h, Ut, preferred_element_type=jnp.float32)
         + jnp.dot(aggr, Ub, preferred_element_type=jnp.float32) + ub)
    mu = jnp.mean(u, axis=0, keepdims=True)
    uc = u - mu
    var = jnp.mean(uc * uc, axis=0, keepdims=True)
    un = uc / jnp.sqrt(var + 1e-5) * g + bb
    return h + _silu(un)


def _tcu_body(h, Sp, invd, hed, W2, b2, Ut, Ub, ub, g, bb, W1n, b1n,
              h_o, P_o):
    hn = _layer_update(h[...], Sp, invd[...], hed[...], W2[...], b2[...],
                       Ut[...], Ub[...], ub[...], g[...], bb[...])
    h_o[...] = hn
    P_o[...] = (jnp.dot(hn, W1n[...], preferred_element_type=jnp.float32)
                + b1n[...])


def _tcu_call(h, Sp, invd, hed, W2, b2, Ut, Ub, ub, g, bb, W1n, b1n):
    f = pl.pallas_call(
        _tcu_body,
        out_shape=[
            jax.ShapeDtypeStruct((N, H), jnp.float32),
            jax.ShapeDtypeStruct((N, H), jnp.float32),
        ],
    )
    return f(h, Sp, invd, hed, W2, b2, Ut, Ub, ub, g, bb, W1n, b1n)


def _tcf_body(h, Sp, invd, hed, W2, b2, Ut, Ub, ub, g, bb,
              prW1, prb1, prW2, prb2, poW1, pob1, poW2, pob2, batch, out_o):
    hn = _layer_update(h[...], Sp, invd[...], hed[...], W2[...], b2[...],
                       Ut[...], Ub[...], ub[...], g[...], bb[...])
    pre = _silu(jnp.dot(hn, prW1[...], preferred_element_type=jnp.float32)
                + prb1[...])
    pre = (jnp.dot(pre, prW2[...], preferred_element_type=jnp.float32)
           + prb2[...])
    gid = lax.broadcasted_iota(jnp.int32, (NG, N), 0)
    oh = (batch[...] == gid).astype(jnp.float32)
    pooled = jnp.dot(oh, pre, preferred_element_type=jnp.float32)
    o = _silu(jnp.dot(pooled, poW1[...], preferred_element_type=jnp.float32)
              + pob1[...])
    out_o[...] = (jnp.dot(o, poW2[...], preferred_element_type=jnp.float32)
                  + pob2[...])


def _tcf_call(h, Sp, invd, hed, W2, b2, Ut, Ub, ub, g, bb,
              prW1, prb1, prW2, prb2, poW1, pob1, poW2, pob2, batch2d):
    f = pl.pallas_call(
        _tcf_body,
        out_shape=jax.ShapeDtypeStruct((NG, H), jnp.float32),
    )
    return f(h, Sp, invd, hed, W2, b2, Ut, Ub, ub, g, bb,
             prW1, prb1, prW2, prb2, poW1, pob1, poW2, pob2, batch2d)


def kernel(x, edge_index, edge_attr, batch, emb_W1, emb_b1, emb_W2, emb_b2,
           msg_W1, msg_b1, msg_W2, msg_b2, upd_W, upd_b, bn_g, bn_b,
           pre_W1, pre_b1, pre_W2, pre_b2, post_W1, post_b1, post_W2,
           post_b2):
    src = edge_index[0]
    dst = edge_index[1]
    ea = edge_attr.reshape(E)
    src_r = src.reshape(NW, NCHUNK, 1, CH)
    dst_r = dst.reshape(NW, NCHUNK, 1, CH)
    ea2 = ea.reshape(E, 1)
    batch2d = batch.reshape(1, N)

    r1 = lambda v: v.reshape(1, -1)

    degr = _deg_call(dst_r)
    h, P, invd, hed = _tc0_call(
        x, emb_W1, r1(emb_b1), emb_W2, r1(emb_b2),
        msg_W1[0, :H, :], r1(msg_b1[0]), degr)

    for l in range(L):
        G = _gather_call(P, src_r)
        sE = _silu_call(G, ea2, msg_W1[l, H, :].reshape(1, H))
        Sp = _scatter_call(sE, dst_r)
        args = (h, Sp, invd, hed, msg_W2[l], r1(msg_b2[l]),
                upd_W[l, :H, :], upd_W[l, H:, :], r1(upd_b[l]),
                r1(bn_g[l]), r1(bn_b[l]))
        if l < L - 1:
            h, P = _tcu_call(*args, msg_W1[l + 1, :H, :], r1(msg_b1[l + 1]))
        else:
            out = _tcf_call(*args, pre_W1, r1(pre_b1), pre_W2, r1(pre_b2),
                            post_W1, r1(post_b1), post_W2, r1(post_b2),
                            batch2d)
    return out


# speedup vs baseline: 2.7132x; 1.0654x over previous
import functools

import jax
import jax.numpy as jnp
from jax import lax
from jax.experimental import pallas as pl
from jax.experimental.pallas import tpu as pltpu
from jax.experimental.pallas import tpu_sc as plsc

N = 10000
E = 320000
H = 128
NG = 64
L = 4

NC = 2
NS = 16
NW = NC * NS
EPT = E // NW
CH = 80
NCHUNK = EPT // CH
RA = 624
TAIL = N - NS * RA
ZR = 104

_MESH = dict(core_axis_name="c", subcore_axis_name="s", num_cores=NC,
             num_subcores=NS)


def _silu(v):
    return v * jax.nn.sigmoid(v)


def _deg_body(dst, out, deg_sh, dst_db, ones_v, semI0, semI1):
    c = lax.axis_index("c")
    s = lax.axis_index("s")
    tile = c * NS + s

    def ob(i, carry):
        for k in range(H // 16):
            ones_v[i, pl.ds(16 * k, 16)] = jnp.ones((16,), jnp.float32)
        return carry
    lax.fori_loop(0, CH, ob, 0)

    def zb(i, carry):
        for k in range(H // 16):
            ones_v[CH + i, pl.ds(16 * k, 16)] = jnp.zeros((16,), jnp.float32)
        return carry
    lax.fori_loop(0, CH, zb, 0)

    rbase = s * RA
    for j in range(7):
        pltpu.sync_copy(ones_v.at[pl.ds(CH, CH)],
                        deg_sh.at[pl.ds(rbase + j * CH, CH)])
    pltpu.sync_copy(ones_v.at[pl.ds(CH, RA - 7 * CH)],
                    deg_sh.at[pl.ds(rbase + 7 * CH, RA - 7 * CH)])

    @pl.when(s == NS - 1)
    def _():
        pltpu.sync_copy(ones_v.at[pl.ds(CH, TAIL)],
                        deg_sh.at[pl.ds(NS * RA, TAIL)])
    plsc.subcore_barrier()

    semI = (semI0, semI1)

    def idx_start(i, b):
        pltpu.async_copy(dst.at[tile, i], dst_db.at[b], semI[b])

    def idx_wait(i, b):
        pltpu.make_async_copy(dst.at[tile, i], dst_db.at[b], semI[b]).wait()

    def proc(b):
        pltpu.sync_copy(ones_v.at[pl.ds(0, CH)], deg_sh.at[dst_db.at[b, 0]],
                        add=True)

    def step(i, b, steady):
        idx_wait(i, b)
        if steady:
            @pl.when(i + 1 < NCHUNK)
            def _():
                idx_start(i + 1, 1 - b)
        proc(b)

    idx_start(0, 0)

    def pair(p_, carry):
        step(2 * p_, 0, True)
        step(2 * p_ + 1, 1, True)
        return carry
    lax.fori_loop(0, NCHUNK // 2, pair, 0)
    step(NCHUNK - 1, 0, False)
    plsc.subcore_barrier()
    pltpu.sync_copy(deg_sh.at[pl.ds(rbase, RA)], out.at[c, pl.ds(rbase, RA)])

    @pl.when(s == NS - 1)
    def _():
        pltpu.sync_copy(deg_sh.at[pl.ds(NS * RA, TAIL)],
                        out.at[c, pl.ds(NS * RA, TAIL)])


def _deg_call(dst):
    f = pl.kernel(
        _deg_body,
        out_type=jax.ShapeDtypeStruct((NC, N, H), jnp.float32),
        mesh=plsc.VectorSubcoreMesh(**_MESH),
        scratch_types=[
            pltpu.VMEM_SHARED((N, H), jnp.float32),
            pltpu.VMEM((2, 1, CH), jnp.int32),
            pltpu.VMEM((2 * CH, H), jnp.float32),
            pltpu.SemaphoreType.DMA,
            pltpu.SemaphoreType.DMA,
        ],
    )
    return f(dst)


def _gather_body(P, src, out, P_sh, src_db, rows_v, semG0, semG1,
                 semI0, semI1):
    c = lax.axis_index("c")
    s = lax.axis_index("s")
    tile = c * NS + s
    semI = (semI0, semI1)
    semG = (semG0, semG1)

    rbase = s * RA
    pltpu.sync_copy(P.at[pl.ds(rbase, RA)], P_sh.at[pl.ds(rbase, RA)])

    @pl.when(s == NS - 1)
    def _():
        pltpu.sync_copy(P.at[pl.ds(NS * RA, TAIL)],
                        P_sh.at[pl.ds(NS * RA, TAIL)])
    plsc.subcore_barrier()

    def idx_start(i, b):
        pltpu.async_copy(src.at[tile, i], src_db.at[b], semI[b])

    def idx_wait(i, b):
        pltpu.make_async_copy(src.at[tile, i], src_db.at[b], semI[b]).wait()

    def g_start(b):
        pltpu.async_copy(P_sh.at[src_db.at[b, 0]], rows_v.at[b], semG[b])

    def g_wait(b):
        pltpu.make_async_copy(P_sh.at[src_db.at[b, 0]], rows_v.at[b],
                              semG[b]).wait()

    def wout(i, b):
        pltpu.sync_copy(rows_v.at[b],
                        out.at[pl.ds(tile * EPT + i * CH, CH)])

    def step(i, b, steady):
        if steady:
            idx_wait(i + 1, 1 - b)
            g_start(1 - b)
        g_wait(b)
        wout(i, b)
        if steady:
            @pl.when(i + 2 < NCHUNK)
            def _():
                idx_start(i + 2, b)

    idx_start(0, 0)
    idx_wait(0, 0)
    g_start(0)
    idx_start(1, 1)

    def pair(p_, carry):
        step(2 * p_, 0, True)
        step(2 * p_ + 1, 1, True)
        return carry
    lax.fori_loop(0, NCHUNK // 2, pair, 0)
    step(NCHUNK - 1, 0, False)


def _gather_call(P, src):
    f = pl.kernel(
        _gather_body,
        out_type=jax.ShapeDtypeStruct((E, H), jnp.float32),
        mesh=plsc.VectorSubcoreMesh(**_MESH),
        scratch_types=[
            pltpu.VMEM_SHARED((N, H), jnp.float32),
            pltpu.VMEM((2, 1, CH), jnp.int32),
            pltpu.VMEM((2, CH, H), jnp.float32),
            pltpu.SemaphoreType.DMA,
            pltpu.SemaphoreType.DMA,
            pltpu.SemaphoreType.DMA,
            pltpu.SemaphoreType.DMA,
        ],
    )
    return f(P, src)


def _scatter_body(sE, dst, out, S_sh, dst_db, rows_v, zbuf, semR0, semR1,
                  semI0, semI1):
    c = lax.axis_index("c")
    s = lax.axis_index("s")
    tile = c * NS + s

    semI = (semI0, semI1)
    semR = (semR0, semR1)

    def idx_start(i, b):
        pltpu.async_copy(dst.at[tile, i], dst_db.at[b], semI[b])

    def idx_wait(i, b):
        pltpu.make_async_copy(dst.at[tile, i], dst_db.at[b], semI[b]).wait()

    def r_start(i, b):
        pltpu.async_copy(sE.at[pl.ds(tile * EPT + i * CH, CH)],
                         rows_v.at[b], semR[b])

    def r_wait(i, b):
        pltpu.make_async_copy(sE.at[pl.ds(tile * EPT + i * CH, CH)],
                              rows_v.at[b], semR[b]).wait()

    def proc(b):
        pltpu.sync_copy(rows_v.at[b], S_sh.at[dst_db.at[b, 0]], add=True)

    def step(i, b, steady):
        idx_wait(i, b)
        r_wait(i, b)
        proc(b)
        if steady:
            @pl.when(i + 2 < NCHUNK)
            def _():
                idx_start(i + 2, b)
                r_start(i + 2, b)

    idx_start(0, 0)
    r_start(0, 0)
    idx_start(1, 1)
    r_start(1, 1)

    def zb(i, carry):
        for k in range(8):
            zbuf[i, pl.ds(16 * k, 16)] = jnp.zeros((16,), jnp.float32)
        return carry
    lax.fori_loop(0, CH, zb, 0)

    rbase = s * RA
    for j in range(7):
        pltpu.sync_copy(zbuf, S_sh.at[pl.ds(rbase + j * CH, CH)])
    pltpu.sync_copy(zbuf.at[pl.ds(0, RA - 7 * CH)],
                    S_sh.at[pl.ds(rbase + 7 * CH, RA - 7 * CH)])

    @pl.when(s == NS - 1)
    def _():
        pltpu.sync_copy(zbuf.at[pl.ds(0, TAIL)],
                        S_sh.at[pl.ds(NS * RA, TAIL)])
    plsc.subcore_barrier()

    def pair(p_, carry):
        step(2 * p_, 0, True)
        step(2 * p_ + 1, 1, True)
        return carry
    lax.fori_loop(0, NCHUNK // 2, pair, 0)
    step(NCHUNK - 1, 0, False)
    plsc.subcore_barrier()
    pltpu.sync_copy(S_sh.at[pl.ds(rbase, RA)], out.at[c, pl.ds(rbase, RA)])

    @pl.when(s == NS - 1)
    def _():
        pltpu.sync_copy(S_sh.at[pl.ds(NS * RA, TAIL)],
                        out.at[c, pl.ds(NS * RA, TAIL)])


def _scatter_call(sE, dst):
    f = pl.kernel(
        _scatter_body,
        out_type=jax.ShapeDtypeStruct((NC, N, H), jnp.float32),
        mesh=plsc.VectorSubcoreMesh(**_MESH),
        scratch_types=[
            pltpu.VMEM_SHARED((N, H), jnp.float32),
            pltpu.VMEM((2, 1, CH), jnp.int32),
            pltpu.VMEM((2, CH, H), jnp.float32),
            pltpu.VMEM((CH, H), jnp.float32),
            pltpu.SemaphoreType.DMA,
            pltpu.SemaphoreType.DMA,
            pltpu.SemaphoreType.DMA,
            pltpu.SemaphoreType.DMA,
        ],
    )
    return f(sE, dst)


BLK = 4000


def _silu_body(G, ea, w1, o):
    v = G[...] + ea[...] * w1[...]
    o[...] = v * jax.nn.sigmoid(v)


def _silu_call(G, ea2, w1row):
    f = pl.pallas_call(
        _silu_body,
        grid=(E // BLK,),
        in_specs=[
            pl.BlockSpec((BLK, H), lambda i: (i, 0)),
            pl.BlockSpec((BLK, 1), lambda i: (i, 0)),
            pl.BlockSpec((1, H), lambda i: (0, 0)),
        ],
        out_specs=pl.BlockSpec((BLK, H), lambda i: (i, 0)),
        out_shape=jax.ShapeDtypeStruct((E, H), jnp.float32),
    )
    return f(G, ea2, w1row)


def _tc0_body(x, eW1, eb1, eW2, eb2, W1h, b1, degr, h_o, P_o, invd_o, hed_o):
    xa = x[...]
    hh = _silu(jnp.dot(xa, eW1[...], preferred_element_type=jnp.float32)
               + eb1[...])
    hh = jnp.dot(hh, eW2[...], preferred_element_type=jnp.float32) + eb2[...]
    h_o[...] = hh
    P_o[...] = (jnp.dot(hh, W1h[...], preferred_element_type=jnp.float32)
                + b1[...])
    cnt = degr[0, :, 0:1] + degr[1, :, 0:1]
    invd_o[...] = 1.0 / jnp.maximum(cnt, 1.0)
    hed_o[...] = jnp.minimum(cnt, 1.0)


def _tc0_call(x, eW1, eb1, eW2, eb2, W1h, b1, degr):
    f = pl.pallas_call(
        _tc0_body,
        out_shape=[
            jax.ShapeDtypeStruct((N, H), jnp.float32),
            jax.ShapeDtypeStruct((N, H), jnp.float32),
            jax.ShapeDtypeStruct((N, 1), jnp.float32),
            jax.ShapeDtypeStruct((N, 1), jnp.float32),
        ],
    )
    return f(x, eW1, eb1, eW2, eb2, W1h, b1, degr)


def _layer_update(h, Sp, invd, hed, W2, b2, Ut, Ub, ub, g, bb):
    S = Sp[0] + Sp[1]
    aggr = (jnp.dot(S, W2, preferred_element_type=jnp.float32) * invd
            + b2 * hed)
    u = (jnp.dot(h, Ut, preferred_element_type=jnp.float32)
         + jnp.dot(aggr, Ub, preferred_element_type=jnp.float32) + ub)
    mu = jnp.mean(u, axis=0, keepdims=True)
    uc = u - mu
    var = jnp.mean(uc * uc, axis=0, keepdims=True)
    un = uc / jnp.sqrt(var + 1e-5) * g + bb
    return h + _silu(un)


def _tcu_body(h, Sp, invd, hed, W2, b2, Ut, Ub, ub, g, bb, W1n, b1n,
              h_o, P_o):
    hn = _layer_update(h[...], Sp, invd[...], hed[...], W2[...], b2[...],
                       Ut[...], Ub[...], ub[...], g[...], bb[...])
    h_o[...] = hn
    P_o[...] = (jnp.dot(hn, W1n[...], preferred_element_type=jnp.float32)
                + b1n[...])


def _tcu_call(h, Sp, invd, hed, W2, b2, Ut, Ub, ub, g, bb, W1n, b1n):
    f = pl.pallas_call(
        _tcu_body,
        out_shape=[
            jax.ShapeDtypeStruct((N, H), jnp.float32),
            jax.ShapeDtypeStruct((N, H), jnp.float32),
        ],
    )
    return f(h, Sp, invd, hed, W2, b2, Ut, Ub, ub, g, bb, W1n, b1n)


def _tcf_body(h, Sp, invd, hed, W2, b2, Ut, Ub, ub, g, bb,
              prW1, prb1, prW2, prb2, poW1, pob1, poW2, pob2, batch, out_o):
    hn = _layer_update(h[...], Sp, invd[...], hed[...], W2[...], b2[...],
                       Ut[...], Ub[...], ub[...], g[...], bb[...])
    pre = _silu(jnp.dot(hn, prW1[...], preferred_element_type=jnp.float32)
                + prb1[...])
    pre = (jnp.dot(pre, prW2[...], preferred_element_type=jnp.float32)
           + prb2[...])
    gid = lax.broadcasted_iota(jnp.int32, (NG, N), 0)
    oh = (batch[...] == gid).astype(jnp.float32)
    pooled = jnp.dot(oh, pre, preferred_element_type=jnp.float32)
    o = _silu(jnp.dot(pooled, poW1[...], preferred_element_type=jnp.float32)
              + pob1[...])
    out_o[...] = (jnp.dot(o, poW2[...], preferred_element_type=jnp.float32)
                  + pob2[...])


def _tcf_call(h, Sp, invd, hed, W2, b2, Ut, Ub, ub, g, bb,
              prW1, prb1, prW2, prb2, poW1, pob1, poW2, pob2, batch2d):
    f = pl.pallas_call(
        _tcf_body,
        out_shape=jax.ShapeDtypeStruct((NG, H), jnp.float32),
    )
    return f(h, Sp, invd, hed, W2, b2, Ut, Ub, ub, g, bb,
             prW1, prb1, prW2, prb2, poW1, pob1, poW2, pob2, batch2d)


def kernel(x, edge_index, edge_attr, batch, emb_W1, emb_b1, emb_W2, emb_b2,
           msg_W1, msg_b1, msg_W2, msg_b2, upd_W, upd_b, bn_g, bn_b,
           pre_W1, pre_b1, pre_W2, pre_b2, post_W1, post_b1, post_W2,
           post_b2):
    src = edge_index[0]
    dst = edge_index[1]
    ea = edge_attr.reshape(E)
    src_r = src.reshape(NW, NCHUNK, 1, CH)
    dst_r = dst.reshape(NW, NCHUNK, 1, CH)
    ea2 = ea.reshape(E, 1)
    batch2d = batch.reshape(1, N)

    r1 = lambda v: v.reshape(1, -1)

    degr = _deg_call(dst_r)
    h, P, invd, hed = _tc0_call(
        x, emb_W1, r1(emb_b1), emb_W2, r1(emb_b2),
        msg_W1[0, :H, :], r1(msg_b1[0]), degr)

    for l in range(L):
        G = _gather_call(P, src_r)
        sE = _silu_call(G, ea2, msg_W1[l, H, :].reshape(1, H))
        Sp = _scatter_call(sE, dst_r)
        args = (h, Sp, invd, hed, msg_W2[l], r1(msg_b2[l]),
                upd_W[l, :H, :], upd_W[l, H:, :], r1(upd_b[l]),
                r1(bn_g[l]), r1(bn_b[l]))
        if l < L - 1:
            h, P = _tcu_call(*args, msg_W1[l + 1, :H, :], r1(msg_b1[l + 1]))
        else:
            out = _tcf_call(*args, pre_W1, r1(pre_b1), pre_W2, r1(pre_b2),
                            post_W1, r1(post_b1), post_W2, r1(post_b2),
                            batch2d)
    return out


# speedup vs baseline: 2.7148x; 1.0006x over previous
import functools

import jax
import jax.numpy as jnp
from jax import lax
from jax.experimental import pallas as pl
from jax.experimental.pallas import tpu as pltpu
from jax.experimental.pallas import tpu_sc as plsc

N = 10000
E = 320000
H = 128
NG = 64
L = 4

NC = 2
NS = 16
NW = NC * NS
EPT = E // NW
CH = 80
NCHUNK = EPT // CH
RA = 624
TAIL = N - NS * RA
ZR = 104

_MESH = dict(core_axis_name="c", subcore_axis_name="s", num_cores=NC,
             num_subcores=NS)


def _silu(v):
    return v * jax.nn.sigmoid(v)


def _deg_body(dst, out, deg_sh, dst_db, ones_v, semI0, semI1):
    c = lax.axis_index("c")
    s = lax.axis_index("s")
    tile = c * NS + s
    semI = (semI0, semI1)

    pltpu.async_copy(dst.at[tile, 0], dst_db.at[0], semI[0])
    pltpu.async_copy(dst.at[tile, 1], dst_db.at[1], semI[1])

    def ob(i, carry):
        for k in range(H // 16):
            ones_v[i, pl.ds(16 * k, 16)] = jnp.ones((16,), jnp.float32)
        return carry
    lax.fori_loop(0, CH, ob, 0)

    def zb(i, carry):
        for k in range(H // 16):
            ones_v[CH + i, pl.ds(16 * k, 16)] = jnp.zeros((16,), jnp.float32)
        return carry
    lax.fori_loop(0, CH, zb, 0)

    rbase = s * RA
    for j in range(7):
        pltpu.sync_copy(ones_v.at[pl.ds(CH, CH)],
                        deg_sh.at[pl.ds(rbase + j * CH, CH)])
    pltpu.sync_copy(ones_v.at[pl.ds(CH, RA - 7 * CH)],
                    deg_sh.at[pl.ds(rbase + 7 * CH, RA - 7 * CH)])

    @pl.when(s == NS - 1)
    def _():
        pltpu.sync_copy(ones_v.at[pl.ds(CH, TAIL)],
                        deg_sh.at[pl.ds(NS * RA, TAIL)])
    plsc.subcore_barrier()

    semI = (semI0, semI1)

    def idx_start(i, b):
        pltpu.async_copy(dst.at[tile, i], dst_db.at[b], semI[b])

    def idx_wait(i, b):
        pltpu.make_async_copy(dst.at[tile, i], dst_db.at[b], semI[b]).wait()

    def proc(b):
        pltpu.sync_copy(ones_v.at[pl.ds(0, CH)], deg_sh.at[dst_db.at[b, 0]],
                        add=True)

    def step(i, b, steady):
        idx_wait(i, b)
        proc(b)
        if steady:
            @pl.when(i + 2 < NCHUNK)
            def _():
                idx_start(i + 2, b)

    def pair(p_, carry):
        step(2 * p_, 0, True)
        step(2 * p_ + 1, 1, True)
        return carry
    lax.fori_loop(0, NCHUNK // 2, pair, 0)
    step(NCHUNK - 1, 0, False)
    plsc.subcore_barrier()
    pltpu.sync_copy(deg_sh.at[pl.ds(rbase, RA)], out.at[c, pl.ds(rbase, RA)])

    @pl.when(s == NS - 1)
    def _():
        pltpu.sync_copy(deg_sh.at[pl.ds(NS * RA, TAIL)],
                        out.at[c, pl.ds(NS * RA, TAIL)])


def _deg_call(dst):
    f = pl.kernel(
        _deg_body,
        out_type=jax.ShapeDtypeStruct((NC, N, H), jnp.float32),
        mesh=plsc.VectorSubcoreMesh(**_MESH),
        scratch_types=[
            pltpu.VMEM_SHARED((N, H), jnp.float32),
            pltpu.VMEM((2, 1, CH), jnp.int32),
            pltpu.VMEM((2 * CH, H), jnp.float32),
            pltpu.SemaphoreType.DMA,
            pltpu.SemaphoreType.DMA,
        ],
    )
    return f(dst)


def _gather_body(P, src, out, P_sh, src_db, rows_v, semG0, semG1,
                 semI0, semI1):
    c = lax.axis_index("c")
    s = lax.axis_index("s")
    tile = c * NS + s
    semI = (semI0, semI1)
    semG = (semG0, semG1)

    def idx_start(i, b):
        pltpu.async_copy(src.at[tile, i], src_db.at[b], semI[b])

    def idx_wait(i, b):
        pltpu.make_async_copy(src.at[tile, i], src_db.at[b], semI[b]).wait()

    idx_start(0, 0)
    idx_start(1, 1)

    rbase = s * RA
    pltpu.sync_copy(P.at[pl.ds(rbase, RA)], P_sh.at[pl.ds(rbase, RA)])

    @pl.when(s == NS - 1)
    def _():
        pltpu.sync_copy(P.at[pl.ds(NS * RA, TAIL)],
                        P_sh.at[pl.ds(NS * RA, TAIL)])
    plsc.subcore_barrier()

    def g_start(b):
        pltpu.async_copy(P_sh.at[src_db.at[b, 0]], rows_v.at[b], semG[b])

    def g_wait(b):
        pltpu.make_async_copy(P_sh.at[src_db.at[b, 0]], rows_v.at[b],
                              semG[b]).wait()

    def wout(i, b):
        pltpu.sync_copy(rows_v.at[b],
                        out.at[pl.ds(tile * EPT + i * CH, CH)])

    def step(i, b, steady):
        if steady:
            idx_wait(i + 1, 1 - b)
            g_start(1 - b)
        g_wait(b)
        wout(i, b)
        if steady:
            @pl.when(i + 2 < NCHUNK)
            def _():
                idx_start(i + 2, b)

    idx_wait(0, 0)
    g_start(0)

    def pair(p_, carry):
        step(2 * p_, 0, True)
        step(2 * p_ + 1, 1, True)
        return carry
    lax.fori_loop(0, NCHUNK // 2, pair, 0)
    step(NCHUNK - 1, 0, False)


def _gather_call(P, src):
    f = pl.kernel(
        _gather_body,
        out_type=jax.ShapeDtypeStruct((E, H), jnp.float32),
        mesh=plsc.VectorSubcoreMesh(**_MESH),
        scratch_types=[
            pltpu.VMEM_SHARED((N, H), jnp.float32),
            pltpu.VMEM((2, 1, CH), jnp.int32),
            pltpu.VMEM((2, CH, H), jnp.float32),
            pltpu.SemaphoreType.DMA,
            pltpu.SemaphoreType.DMA,
            pltpu.SemaphoreType.DMA,
            pltpu.SemaphoreType.DMA,
        ],
    )
    return f(P, src)


def _scatter_body(sE, dst, out, S_sh, dst_db, rows_v, zbuf, semR0, semR1,
                  semI0, semI1):
    c = lax.axis_index("c")
    s = lax.axis_index("s")
    tile = c * NS + s

    semI = (semI0, semI1)
    semR = (semR0, semR1)

    def idx_start(i, b):
        pltpu.async_copy(dst.at[tile, i], dst_db.at[b], semI[b])

    def idx_wait(i, b):
        pltpu.make_async_copy(dst.at[tile, i], dst_db.at[b], semI[b]).wait()

    def r_start(i, b):
        pltpu.async_copy(sE.at[pl.ds(tile * EPT + i * CH, CH)],
                         rows_v.at[b], semR[b])

    def r_wait(i, b):
        pltpu.make_async_copy(sE.at[pl.ds(tile * EPT + i * CH, CH)],
                              rows_v.at[b], semR[b]).wait()

    def proc(b):
        pltpu.sync_copy(rows_v.at[b], S_sh.at[dst_db.at[b, 0]], add=True)

    def step(i, b, steady):
        idx_wait(i, b)
        r_wait(i, b)
        proc(b)
        if steady:
            @pl.when(i + 2 < NCHUNK)
            def _():
                idx_start(i + 2, b)
                r_start(i + 2, b)

    idx_start(0, 0)
    r_start(0, 0)
    idx_start(1, 1)
    r_start(1, 1)

    def zb(i, carry):
        for k in range(8):
            zbuf[i, pl.ds(16 * k, 16)] = jnp.zeros((16,), jnp.float32)
        return carry
    lax.fori_loop(0, CH, zb, 0)

    rbase = s * RA
    for j in range(7):
        pltpu.sync_copy(zbuf, S_sh.at[pl.ds(rbase + j * CH, CH)])
    pltpu.sync_copy(zbuf.at[pl.ds(0, RA - 7 * CH)],
                    S_sh.at[pl.ds(rbase + 7 * CH, RA - 7 * CH)])

    @pl.when(s == NS - 1)
    def _():
        pltpu.sync_copy(zbuf.at[pl.ds(0, TAIL)],
                        S_sh.at[pl.ds(NS * RA, TAIL)])
    plsc.subcore_barrier()

    def pair(p_, carry):
        step(2 * p_, 0, True)
        step(2 * p_ + 1, 1, True)
        return carry
    lax.fori_loop(0, NCHUNK // 2, pair, 0)
    step(NCHUNK - 1, 0, False)
    plsc.subcore_barrier()
    pltpu.sync_copy(S_sh.at[pl.ds(rbase, RA)], out.at[c, pl.ds(rbase, RA)])

    @pl.when(s == NS - 1)
    def _():
        pltpu.sync_copy(S_sh.at[pl.ds(NS * RA, TAIL)],
                        out.at[c, pl.ds(NS * RA, TAIL)])


def _scatter_call(sE, dst):
    f = pl.kernel(
        _scatter_body,
        out_type=jax.ShapeDtypeStruct((NC, N, H), jnp.float32),
        mesh=plsc.VectorSubcoreMesh(**_MESH),
        scratch_types=[
            pltpu.VMEM_SHARED((N, H), jnp.float32),
            pltpu.VMEM((2, 1, CH), jnp.int32),
            pltpu.VMEM((2, CH, H), jnp.float32),
            pltpu.VMEM((CH, H), jnp.float32),
            pltpu.SemaphoreType.DMA,
            pltpu.SemaphoreType.DMA,
            pltpu.SemaphoreType.DMA,
            pltpu.SemaphoreType.DMA,
        ],
    )
    return f(sE, dst)


BLK = 4000


def _silu_body(G, ea, w1, o):
    v = G[...] + ea[...] * w1[...]
    o[...] = v * jax.nn.sigmoid(v)


def _silu_call(G, ea2, w1row):
    f = pl.pallas_call(
        _silu_body,
        grid=(E // BLK,),
        in_specs=[
            pl.BlockSpec((BLK, H), lambda i: (i, 0)),
            pl.BlockSpec((BLK, 1), lambda i: (i, 0)),
            pl.BlockSpec((1, H), lambda i: (0, 0)),
        ],
        out_specs=pl.BlockSpec((BLK, H), lambda i: (i, 0)),
        out_shape=jax.ShapeDtypeStruct((E, H), jnp.float32),
    )
    return f(G, ea2, w1row)


def _tc0_body(x, eW1, eb1, eW2, eb2, W1h, b1, degr, h_o, P_o, invd_o, hed_o):
    xa = x[...]
    hh = _silu(jnp.dot(xa, eW1[...], preferred_element_type=jnp.float32)
               + eb1[...])
    hh = jnp.dot(hh, eW2[...], preferred_element_type=jnp.float32) + eb2[...]
    h_o[...] = hh
    P_o[...] = (jnp.dot(hh, W1h[...], preferred_element_type=jnp.float32)
                + b1[...])
    cnt = degr[0, :, 0:1] + degr[1, :, 0:1]
    invd_o[...] = 1.0 / jnp.maximum(cnt, 1.0)
    hed_o[...] = jnp.minimum(cnt, 1.0)


def _tc0_call(x, eW1, eb1, eW2, eb2, W1h, b1, degr):
    f = pl.pallas_call(
        _tc0_body,
        out_shape=[
            jax.ShapeDtypeStruct((N, H), jnp.float32),
            jax.ShapeDtypeStruct((N, H), jnp.float32),
            jax.ShapeDtypeStruct((N, 1), jnp.float32),
            jax.ShapeDtypeStruct((N, 1), jnp.float32),
        ],
    )
    return f(x, eW1, eb1, eW2, eb2, W1h, b1, degr)


def _layer_update(h, Sp, invd, hed, W2, b2, Ut, Ub, ub, g, bb):
    S = Sp[0] + Sp[1]
    aggr = (jnp.dot(S, W2, preferred_element_type=jnp.float32) * invd
            + b2 * hed)
    u = (jnp.dot(h, Ut, preferred_element_type=jnp.float32)
         + jnp.dot(aggr, Ub, preferred_element_type=jnp.float32) + ub)
    mu = jnp.mean(u, axis=0, keepdims=True)
    uc = u - mu
    var = jnp.mean(uc * uc, axis=0, keepdims=True)
    un = uc / jnp.sqrt(var + 1e-5) * g + bb
    return h + _silu(un)


def _tcu_body(h, Sp, invd, hed, W2, b2, Ut, Ub, ub, g, bb, W1n, b1n,
              h_o, P_o):
    hn = _layer_update(h[...], Sp, invd[...], hed[...], W2[...], b2[...],
                       Ut[...], Ub[...], ub[...], g[...], bb[...])
    h_o[...] = hn
    P_o[...] = (jnp.dot(hn, W1n[...], preferred_element_type=jnp.float32)
                + b1n[...])


def _tcu_call(h, Sp, invd, hed, W2, b2, Ut, Ub, ub, g, bb, W1n, b1n):
    f = pl.pallas_call(
        _tcu_body,
        out_shape=[
            jax.ShapeDtypeStruct((N, H), jnp.float32),
            jax.ShapeDtypeStruct((N, H), jnp.float32),
        ],
    )
    return f(h, Sp, invd, hed, W2, b2, Ut, Ub, ub, g, bb, W1n, b1n)


def _tcf_body(h, Sp, invd, hed, W2, b2, Ut, Ub, ub, g, bb,
              prW1, prb1, prW2, prb2, poW1, pob1, poW2, pob2, batch, out_o):
    hn = _layer_update(h[...], Sp, invd[...], hed[...], W2[...], b2[...],
                       Ut[...], Ub[...], ub[...], g[...], bb[...])
    pre = _silu(jnp.dot(hn, prW1[...], preferred_element_type=jnp.float32)
                + prb1[...])
    pre = (jnp.dot(pre, prW2[...], preferred_element_type=jnp.float32)
           + prb2[...])
    gid = lax.broadcasted_iota(jnp.int32, (NG, N), 0)
    oh = (batch[...] == gid).astype(jnp.float32)
    pooled = jnp.dot(oh, pre, preferred_element_type=jnp.float32)
    o = _silu(jnp.dot(pooled, poW1[...], preferred_element_type=jnp.float32)
              + pob1[...])
    out_o[...] = (jnp.dot(o, poW2[...], preferred_element_type=jnp.float32)
                  + pob2[...])


def _tcf_call(h, Sp, invd, hed, W2, b2, Ut, Ub, ub, g, bb,
              prW1, prb1, prW2, prb2, poW1, pob1, poW2, pob2, batch2d):
    f = pl.pallas_call(
        _tcf_body,
        out_shape=jax.ShapeDtypeStruct((NG, H), jnp.float32),
    )
    return f(h, Sp, invd, hed, W2, b2, Ut, Ub, ub, g, bb,
             prW1, prb1, prW2, prb2, poW1, pob1, poW2, pob2, batch2d)


def kernel(x, edge_index, edge_attr, batch, emb_W1, emb_b1, emb_W2, emb_b2,
           msg_W1, msg_b1, msg_W2, msg_b2, upd_W, upd_b, bn_g, bn_b,
           pre_W1, pre_b1, pre_W2, pre_b2, post_W1, post_b1, post_W2,
           post_b2):
    src = edge_index[0]
    dst = edge_index[1]
    ea = edge_attr.reshape(E)
    src_r = src.reshape(NW, NCHUNK, 1, CH)
    dst_r = dst.reshape(NW, NCHUNK, 1, CH)
    ea2 = ea.reshape(E, 1)
    batch2d = batch.reshape(1, N)

    r1 = lambda v: v.reshape(1, -1)

    degr = _deg_call(dst_r)
    h, P, invd, hed = _tc0_call(
        x, emb_W1, r1(emb_b1), emb_W2, r1(emb_b2),
        msg_W1[0, :H, :], r1(msg_b1[0]), degr)

    for l in range(L):
        G = _gather_call(P, src_r)
        sE = _silu_call(G, ea2, msg_W1[l, H, :].reshape(1, H))
        Sp = _scatter_call(sE, dst_r)
        args = (h, Sp, invd, hed, msg_W2[l], r1(msg_b2[l]),
                upd_W[l, :H, :], upd_W[l, H:, :], r1(upd_b[l]),
                r1(bn_g[l]), r1(bn_b[l]))
        if l < L - 1:
            h, P = _tcu_call(*args, msg_W1[l + 1, :H, :], r1(msg_b1[l + 1]))
        else:
            out = _tcf_call(*args, pre_W1, r1(pre_b1), pre_W2, r1(pre_b2),
                            post_W1, r1(post_b1), post_W2, r1(post_b2),
                            batch2d)
    return out


# speedup vs baseline: 2.7171x; 1.0008x over previous
import jax
import jax.numpy as jnp
from jax import lax
from jax.experimental import pallas as pl
from jax.experimental.pallas import tpu as pltpu
from jax.experimental.pallas import tpu_sc as plsc

N = 10000
E = 320000
H = 128
NG = 64
L = 4

NC = 2
NS = 16
NW = NC * NS
EPT = E // NW
CH = 80
NCHUNK = EPT // CH
RA = 624
TAIL = N - NS * RA

_MESH = dict(core_axis_name="c", subcore_axis_name="s", num_cores=NC,
             num_subcores=NS)


def _silu(v):
    return v * jax.nn.sigmoid(v)


def _deg_body(dst, out, deg_sh, dst_db, ones_v, semI0, semI1):
    c = lax.axis_index("c")
    s = lax.axis_index("s")
    tile = c * NS + s
    semI = (semI0, semI1)

    pltpu.async_copy(dst.at[tile, 0], dst_db.at[0], semI[0])
    pltpu.async_copy(dst.at[tile, 1], dst_db.at[1], semI[1])

    def ob(i, carry):
        for k in range(H // 16):
            ones_v[i, pl.ds(16 * k, 16)] = jnp.ones((16,), jnp.float32)
        return carry
    lax.fori_loop(0, CH, ob, 0)

    def zb(i, carry):
        for k in range(H // 16):
            ones_v[CH + i, pl.ds(16 * k, 16)] = jnp.zeros((16,), jnp.float32)
        return carry
    lax.fori_loop(0, CH, zb, 0)

    rbase = s * RA
    for j in range(7):
        pltpu.sync_copy(ones_v.at[pl.ds(CH, CH)],
                        deg_sh.at[pl.ds(rbase + j * CH, CH)])
    pltpu.sync_copy(ones_v.at[pl.ds(CH, RA - 7 * CH)],
                    deg_sh.at[pl.ds(rbase + 7 * CH, RA - 7 * CH)])

    @pl.when(s == NS - 1)
    def _():
        pltpu.sync_copy(ones_v.at[pl.ds(CH, TAIL)],
                        deg_sh.at[pl.ds(NS * RA, TAIL)])
    plsc.subcore_barrier()

    semI = (semI0, semI1)

    def idx_start(i, b):
        pltpu.async_copy(dst.at[tile, i], dst_db.at[b], semI[b])

    def idx_wait(i, b):
        pltpu.make_async_copy(dst.at[tile, i], dst_db.at[b], semI[b]).wait()

    def proc(b):
        pltpu.sync_copy(ones_v.at[pl.ds(0, CH)], deg_sh.at[dst_db.at[b, 0]],
                        add=True)

    def step(i, b, steady):
        idx_wait(i, b)
        proc(b)
        if steady:
            @pl.when(i + 2 < NCHUNK)
            def _():
                idx_start(i + 2, b)

    def pair(p_, carry):
        step(2 * p_, 0, True)
        step(2 * p_ + 1, 1, True)
        return carry
    lax.fori_loop(0, NCHUNK // 2, pair, 0)
    step(NCHUNK - 1, 0, False)
    plsc.subcore_barrier()
    pltpu.sync_copy(deg_sh.at[pl.ds(rbase, RA)], out.at[c, pl.ds(rbase, RA)])

    @pl.when(s == NS - 1)
    def _():
        pltpu.sync_copy(deg_sh.at[pl.ds(NS * RA, TAIL)],
                        out.at[c, pl.ds(NS * RA, TAIL)])


def _deg_call(dst):
    f = pl.kernel(
        _deg_body,
        out_type=jax.ShapeDtypeStruct((NC, N, H), jnp.float32),
        mesh=plsc.VectorSubcoreMesh(**_MESH),
        scratch_types=[
            pltpu.VMEM_SHARED((N, H), jnp.float32),
            pltpu.VMEM((2, 1, CH), jnp.int32),
            pltpu.VMEM((2 * CH, H), jnp.float32),
            pltpu.SemaphoreType.DMA,
            pltpu.SemaphoreType.DMA,
        ],
    )
    return f(dst)


def _gather_body(P, src, out, P_sh, src_db, rows_v, semG0, semG1,
                 semI0, semI1):
    c = lax.axis_index("c")
    s = lax.axis_index("s")
    tile = c * NS + s
    semI = (semI0, semI1)
    semG = (semG0, semG1)

    def idx_start(i, b):
        pltpu.async_copy(src.at[tile, i], src_db.at[b], semI[b])

    def idx_wait(i, b):
        pltpu.make_async_copy(src.at[tile, i], src_db.at[b], semI[b]).wait()

    idx_start(0, 0)
    idx_start(1, 1)

    rbase = s * RA
    pltpu.sync_copy(P.at[pl.ds(rbase, RA)], P_sh.at[pl.ds(rbase, RA)])

    @pl.when(s == NS - 1)
    def _():
        pltpu.sync_copy(P.at[pl.ds(NS * RA, TAIL)],
                        P_sh.at[pl.ds(NS * RA, TAIL)])
    plsc.subcore_barrier()

    def g_start(b):
        pltpu.async_copy(P_sh.at[src_db.at[b, 0]], rows_v.at[b], semG[b])

    def g_wait(b):
        pltpu.make_async_copy(P_sh.at[src_db.at[b, 0]], rows_v.at[b],
                              semG[b]).wait()

    def wout(i, b):
        pltpu.sync_copy(rows_v.at[b],
                        out.at[pl.ds(tile * EPT + i * CH, CH)])

    def step(i, b, steady):
        if steady:
            idx_wait(i + 1, 1 - b)
            g_start(1 - b)
        g_wait(b)
        wout(i, b)
        if steady:
            @pl.when(i + 2 < NCHUNK)
            def _():
                idx_start(i + 2, b)

    idx_wait(0, 0)
    g_start(0)

    def pair(p_, carry):
        step(2 * p_, 0, True)
        step(2 * p_ + 1, 1, True)
        return carry
    lax.fori_loop(0, NCHUNK // 2, pair, 0)
    step(NCHUNK - 1, 0, False)


def _gather_call(P, src):
    f = pl.kernel(
        _gather_body,
        out_type=jax.ShapeDtypeStruct((E, H), jnp.float32),
        mesh=plsc.VectorSubcoreMesh(**_MESH),
        scratch_types=[
            pltpu.VMEM_SHARED((N, H), jnp.float32),
            pltpu.VMEM((2, 1, CH), jnp.int32),
            pltpu.VMEM((2, CH, H), jnp.float32),
            pltpu.SemaphoreType.DMA,
            pltpu.SemaphoreType.DMA,
            pltpu.SemaphoreType.DMA,
            pltpu.SemaphoreType.DMA,
        ],
    )
    return f(P, src)


def _scatter_body(sE, dst, out, S_sh, dst_db, rows_v, zbuf, semR0, semR1,
                  semI0, semI1):
    c = lax.axis_index("c")
    s = lax.axis_index("s")
    tile = c * NS + s

    semI = (semI0, semI1)
    semR = (semR0, semR1)

    def idx_start(i, b):
        pltpu.async_copy(dst.at[tile, i], dst_db.at[b], semI[b])

    def idx_wait(i, b):
        pltpu.make_async_copy(dst.at[tile, i], dst_db.at[b], semI[b]).wait()

    def r_start(i, b):
        pltpu.async_copy(sE.at[pl.ds(tile * EPT + i * CH, CH)],
                         rows_v.at[b], semR[b])

    def r_wait(i, b):
        pltpu.make_async_copy(sE.at[pl.ds(tile * EPT + i * CH, CH)],
                              rows_v.at[b], semR[b]).wait()

    def proc(b):
        pltpu.sync_copy(rows_v.at[b], S_sh.at[dst_db.at[b, 0]], add=True)

    def step(i, b, steady):
        idx_wait(i, b)
        r_wait(i, b)
        proc(b)
        if steady:
            @pl.when(i + 2 < NCHUNK)
            def _():
                idx_start(i + 2, b)
                r_start(i + 2, b)

    idx_start(0, 0)
    r_start(0, 0)
    idx_start(1, 1)
    r_start(1, 1)

    def zb(i, carry):
        for k in range(8):
            zbuf[i, pl.ds(16 * k, 16)] = jnp.zeros((16,), jnp.float32)
        return carry
    lax.fori_loop(0, CH, zb, 0)

    rbase = s * RA
    for j in range(7):
        pltpu.sync_copy(zbuf, S_sh.at[pl.ds(rbase + j * CH, CH)])
    pltpu.sync_copy(zbuf.at[pl.ds(0, RA - 7 * CH)],
                    S_sh.at[pl.ds(rbase + 7 * CH, RA - 7 * CH)])

    @pl.when(s == NS - 1)
    def _():
        pltpu.sync_copy(zbuf.at[pl.ds(0, TAIL)],
                        S_sh.at[pl.ds(NS * RA, TAIL)])
    plsc.subcore_barrier()

    def pair(p_, carry):
        step(2 * p_, 0, True)
        step(2 * p_ + 1, 1, True)
        return carry
    lax.fori_loop(0, NCHUNK // 2, pair, 0)
    step(NCHUNK - 1, 0, False)
    plsc.subcore_barrier()
    pltpu.sync_copy(S_sh.at[pl.ds(rbase, RA)], out.at[c, pl.ds(rbase, RA)])

    @pl.when(s == NS - 1)
    def _():
        pltpu.sync_copy(S_sh.at[pl.ds(NS * RA, TAIL)],
                        out.at[c, pl.ds(NS * RA, TAIL)])


def _scatter_call(sE, dst):
    f = pl.kernel(
        _scatter_body,
        out_type=jax.ShapeDtypeStruct((NC, N, H), jnp.float32),
        mesh=plsc.VectorSubcoreMesh(**_MESH),
        scratch_types=[
            pltpu.VMEM_SHARED((N, H), jnp.float32),
            pltpu.VMEM((2, 1, CH), jnp.int32),
            pltpu.VMEM((2, CH, H), jnp.float32),
            pltpu.VMEM((CH, H), jnp.float32),
            pltpu.SemaphoreType.DMA,
            pltpu.SemaphoreType.DMA,
            pltpu.SemaphoreType.DMA,
            pltpu.SemaphoreType.DMA,
        ],
    )
    return f(sE, dst)


BLK = 4000


def _silu_body(G, ea, w1, o):
    v = G[...] + ea[...] * w1[...]
    o[...] = v * jax.nn.sigmoid(v)


def _silu_call(G, ea2, w1row):
    f = pl.pallas_call(
        _silu_body,
        grid=(E // BLK,),
        in_specs=[
            pl.BlockSpec((BLK, H), lambda i: (i, 0)),
            pl.BlockSpec((BLK, 1), lambda i: (i, 0)),
            pl.BlockSpec((1, H), lambda i: (0, 0)),
        ],
        out_specs=pl.BlockSpec((BLK, H), lambda i: (i, 0)),
        out_shape=jax.ShapeDtypeStruct((E, H), jnp.float32),
    )
    return f(G, ea2, w1row)


def _tc0_body(x, eW1, eb1, eW2, eb2, W1h, b1, degr, h_o, P_o, invd_o, hed_o):
    xa = x[...]
    hh = _silu(jnp.dot(xa, eW1[...], preferred_element_type=jnp.float32)
               + eb1[...])
    hh = jnp.dot(hh, eW2[...], preferred_element_type=jnp.float32) + eb2[...]
    h_o[...] = hh
    P_o[...] = (jnp.dot(hh, W1h[...], preferred_element_type=jnp.float32)
                + b1[...])
    cnt = degr[0, :, 0:1] + degr[1, :, 0:1]
    invd_o[...] = 1.0 / jnp.maximum(cnt, 1.0)
    hed_o[...] = jnp.minimum(cnt, 1.0)


def _tc0_call(x, eW1, eb1, eW2, eb2, W1h, b1, degr):
    f = pl.pallas_call(
        _tc0_body,
        out_shape=[
            jax.ShapeDtypeStruct((N, H), jnp.float32),
            jax.ShapeDtypeStruct((N, H), jnp.float32),
            jax.ShapeDtypeStruct((N, 1), jnp.float32),
            jax.ShapeDtypeStruct((N, 1), jnp.float32),
        ],
    )
    return f(x, eW1, eb1, eW2, eb2, W1h, b1, degr)


def _layer_update(h, Sp, invd, hed, W2, b2, Ut, Ub, ub, g, bb):
    S = Sp[0] + Sp[1]
    aggr = (jnp.dot(S, W2, preferred_element_type=jnp.float32) * invd
            + b2 * hed)
    u = (jnp.dot(h, Ut, preferred_element_type=jnp.float32)
         + jnp.dot(aggr, Ub, preferred_element_type=jnp.float32) + ub)
    mu = jnp.mean(u, axis=0, keepdims=True)
    uc = u - mu
    var = jnp.mean(uc * uc, axis=0, keepdims=True)
    un = uc / jnp.sqrt(var + 1e-5) * g + bb
    return h + _silu(un)


def _tcu_body(h, Sp, invd, hed, W2, b2, Ut, Ub, ub, g, bb, W1n, b1n,
              h_o, P_o):
    hn = _layer_update(h[...], Sp, invd[...], hed[...], W2[...], b2[...],
                       Ut[...], Ub[...], ub[...], g[...], bb[...])
    h_o[...] = hn
    P_o[...] = (jnp.dot(hn, W1n[...], preferred_element_type=jnp.float32)
                + b1n[...])


def _tcu_call(h, Sp, invd, hed, W2, b2, Ut, Ub, ub, g, bb, W1n, b1n):
    f = pl.pallas_call(
        _tcu_body,
        out_shape=[
            jax.ShapeDtypeStruct((N, H), jnp.float32),
            jax.ShapeDtypeStruct((N, H), jnp.float32),
        ],
    )
    return f(h, Sp, invd, hed, W2, b2, Ut, Ub, ub, g, bb, W1n, b1n)


def _tcf_body(h, Sp, invd, hed, W2, b2, Ut, Ub, ub, g, bb,
              prW1, prb1, prW2, prb2, poW1, pob1, poW2, pob2, batch, out_o):
    hn = _layer_update(h[...], Sp, invd[...], hed[...], W2[...], b2[...],
                       Ut[...], Ub[...], ub[...], g[...], bb[...])
    pre = _silu(jnp.dot(hn, prW1[...], preferred_element_type=jnp.float32)
                + prb1[...])
    pre = (jnp.dot(pre, prW2[...], preferred_element_type=jnp.float32)
           + prb2[...])
    gid = lax.broadcasted_iota(jnp.int32, (NG, N), 0)
    oh = (batch[...] == gid).astype(jnp.float32)
    pooled = jnp.dot(oh, pre, preferred_element_type=jnp.float32)
    o = _silu(jnp.dot(pooled, poW1[...], preferred_element_type=jnp.float32)
              + pob1[...])
    out_o[...] = (jnp.dot(o, poW2[...], preferred_element_type=jnp.float32)
                  + pob2[...])


def _tcf_call(h, Sp, invd, hed, W2, b2, Ut, Ub, ub, g, bb,
              prW1, prb1, prW2, prb2, poW1, pob1, poW2, pob2, batch2d):
    f = pl.pallas_call(
        _tcf_body,
        out_shape=jax.ShapeDtypeStruct((NG, H), jnp.float32),
    )
    return f(h, Sp, invd, hed, W2, b2, Ut, Ub, ub, g, bb,
             prW1, prb1, prW2, prb2, poW1, pob1, poW2, pob2, batch2d)


def kernel(x, edge_index, edge_attr, batch, emb_W1, emb_b1, emb_W2, emb_b2,
           msg_W1, msg_b1, msg_W2, msg_b2, upd_W, upd_b, bn_g, bn_b,
           pre_W1, pre_b1, pre_W2, pre_b2, post_W1, post_b1, post_W2,
           post_b2):
    src = edge_index[0]
    dst = edge_index[1]
    ea = edge_attr.reshape(E)
    src_r = src.reshape(NW, NCHUNK, 1, CH)
    dst_r = dst.reshape(NW, NCHUNK, 1, CH)
    ea2 = ea.reshape(E, 1)
    batch2d = batch.reshape(1, N)

    r1 = lambda v: v.reshape(1, -1)

    degr = _deg_call(dst_r)
    h, P, invd, hed = _tc0_call(
        x, emb_W1, r1(emb_b1), emb_W2, r1(emb_b2),
        msg_W1[0, :H, :], r1(msg_b1[0]), degr)

    for l in range(L):
        G = _gather_call(P, src_r)
        sE = _silu_call(G, ea2, msg_W1[l, H, :].reshape(1, H))
        Sp = _scatter_call(sE, dst_r)
        args = (h, Sp, invd, hed, msg_W2[l], r1(msg_b2[l]),
                upd_W[l, :H, :], upd_W[l, H:, :], r1(upd_b[l]),
                r1(bn_g[l]), r1(bn_b[l]))
        if l < L - 1:
            h, P = _tcu_call(*args, msg_W1[l + 1, :H, :], r1(msg_b1[l + 1]))
        else:
            out = _tcf_call(*args, pre_W1, r1(pre_b1), pre_W2, r1(pre_b2),
                            post_W1, r1(post_b1), post_W2, r1(post_b2),
                            batch2d)
    return out
